# Initial kernel scaffold; baseline (speedup 1.0000x reference)
import jax
import jax.numpy as jnp
from jax.experimental import pallas as pl


def kernel(x, edge_index, batch, W1, b1, g1, be1, W2, b2, g2, be2, W3, b3, g3, be3, Wfc, bfc):
    raise NotImplementedError("write your pallas kernel here")


# speedup vs baseline: 11.7720x; 11.7720x over previous
import functools

import jax
import jax.numpy as jnp
from jax import lax
from jax.experimental import pallas as pl
from jax.experimental.pallas import tpu as pltpu
from jax.experimental.pallas import tpu_sc as plsc

_N = 10000
_E = 320000
_D = 128
_H = 64
_G = 64

_NW = 32
_K = 128
_EP = 323584
_CHUNKS = _EP // (_NW * _K)
_NP = 10112
_RPT = _NP // 16
_DW = 8

_mesh = plsc.VectorSubcoreMesh(core_axis_name="c", subcore_axis_name="s")


@functools.partial(
    pl.kernel,
    mesh=_mesh,
    compiler_params=pltpu.CompilerParams(use_tc_tiling_on_sc=False),
    out_type=jax.ShapeDtypeStruct((2, _NP, _DW), jnp.float32),
    scratch_types=[
        pltpu.VMEM((_K,), jnp.int32),
        pltpu.VMEM((_K, _DW), jnp.float32),
        pltpu.VMEM_SHARED((_NP, _DW), jnp.float32),
    ],
)
def _deg_pass(dst_hbm, ones_hbm, zrow_hbm, out_hbm, didx_v, ones_v, deg_sh):
    c = lax.axis_index("c")
    s = lax.axis_index("s")
    pltpu.sync_copy(zrow_hbm, deg_sh.at[pl.ds(s * _RPT, _RPT)])
    pltpu.sync_copy(ones_hbm, ones_v)
    plsc.subcore_barrier()
    ebase = c * (_EP // 2) + s * (_EP // _NW)

    def body(i, carry):
        base = ebase + i * _K
        pltpu.sync_copy(dst_hbm.at[pl.ds(base, _K)], didx_v)
        pltpu.sync_copy(ones_v, deg_sh.at[didx_v], add=True)
        return carry

    lax.fori_loop(0, _CHUNKS, body, 0)
    plsc.subcore_barrier()
    pltpu.sync_copy(deg_sh.at[pl.ds(s * _RPT, _RPT)],
                    out_hbm.at[c, pl.ds(s * _RPT, _RPT)])


@functools.partial(
    pl.kernel,
    mesh=_mesh,
    compiler_params=pltpu.CompilerParams(use_tc_tiling_on_sc=False),
    out_type=jax.ShapeDtypeStruct((2, _NP, _H), jnp.float32),
    scratch_types=[
        pltpu.VMEM((_K,), jnp.int32),
        pltpu.VMEM((_K,), jnp.int32),
        pltpu.VMEM((_K, _H), jnp.float32),
        pltpu.VMEM_SHARED((_NP, _H), jnp.float32),
        pltpu.SemaphoreType.DMA,
    ],
)
def _seg_pass(a_hbm, src_hbm, dst_hbm, zrow_hbm, out_hbm,
              sidx_v, didx_v, rows_v, agg_sh, sem):
    c = lax.axis_index("c")
    s = lax.axis_index("s")
    pltpu.sync_copy(zrow_hbm, agg_sh.at[pl.ds(s * _RPT, _RPT)])
    plsc.subcore_barrier()
    ebase = c * (_EP // 2) + s * (_EP // _NW)

    def body(i, carry):
        base = ebase + i * _K
        pltpu.sync_copy(src_hbm.at[pl.ds(base, _K)], sidx_v)
        pltpu.async_copy(a_hbm.at[sidx_v], rows_v, sem).wait()
        pltpu.sync_copy(dst_hbm.at[pl.ds(base, _K)], didx_v)
        pltpu.sync_copy(rows_v, agg_sh.at[didx_v], add=True)
        return carry

    lax.fori_loop(0, _CHUNKS, body, 0)
    plsc.subcore_barrier()
    pltpu.sync_copy(agg_sh.at[pl.ds(s * _RPT, _RPT)],
                    out_hbm.at[c, pl.ds(s * _RPT, _RPT)])


_RB = 2000
_NBLK = _N // _RB


def _prelude_body(x_ref, w_ref, dp_ref, dinv_ref, a_ref):
    deg = dp_ref[0] + dp_ref[1] + 1.0
    dinv = lax.rsqrt(deg)
    dinv_ref[...] = dinv
    xw = jnp.dot(x_ref[...], w_ref[...], preferred_element_type=jnp.float32,
                 precision=lax.Precision.HIGHEST)
    a_ref[...] = dinv * xw


def _conv_body(p_ref, a_ref, dinv_ref, b_ref, conv_ref):
    conv_ref[...] = (dinv_ref[...] * (p_ref[0] + p_ref[1] + a_ref[...])
                     + b_ref[...])


def _stats_body(c_ref, m_ref, v_ref):
    c = c_ref[...]
    m = jnp.mean(c, axis=0, keepdims=True)
    m_ref[...] = m
    v_ref[...] = jnp.mean((c - m) * (c - m), axis=0, keepdims=True)


def _norm_body(has_res, c_ref, m_ref, v_ref, gm_ref, bt_ref, *rest):
    if has_res:
        res_ref, w_ref, dinv_ref, h_ref, an_ref = rest
    else:
        w_ref, dinv_ref, h_ref, an_ref = rest
    h = ((c_ref[...] - m_ref[...]) / jnp.sqrt(v_ref[...] + 1e-5)
         * gm_ref[...] + bt_ref[...])
    if has_res:
        h = h + res_ref[...]
    h = jnp.maximum(h, 0.0)
    h_ref[...] = h
    hw = jnp.dot(h, w_ref[...], preferred_element_type=jnp.float32,
                 precision=lax.Precision.HIGHEST)
    an_ref[...] = dinv_ref[...] * hw


def _pool_body(c_ref, m_ref, v_ref, gm_ref, bt_ref, batch_ref,
               wfc_ref, bfc_ref, out_ref):
    h = ((c_ref[...] - m_ref[...]) / jnp.sqrt(v_ref[...] + 1e-5)
         * gm_ref[...] + bt_ref[...])
    h = jnp.maximum(h, 0.0)
    gids = lax.broadcasted_iota(jnp.int32, (_G, _N), 0)
    onehot = (gids == batch_ref[...]).astype(jnp.float32)
    counts = jnp.sum(onehot, axis=1, keepdims=True)
    pooled = jnp.dot(onehot, h, preferred_element_type=jnp.float32,
                     precision=lax.Precision.HIGHEST)
    pooled = pooled / jnp.maximum(counts, 1.0)
    out_ref[...] = jnp.dot(pooled, wfc_ref[...],
                           preferred_element_type=jnp.float32,
                           precision=lax.Precision.HIGHEST) + bfc_ref[...]


def _rows(bs):
    return pl.BlockSpec((_RB, bs), lambda i: (i, 0))


def _bcast(r, c):
    return pl.BlockSpec((r, c), lambda i: (0, 0))


def _pad_rows(a):
    return jnp.concatenate([a, jnp.zeros((_NP - _N, _H), jnp.float32)], axis=0)


def _conv(p, a, dinv, b):
    return pl.pallas_call(
        _conv_body,
        grid=(_NBLK,),
        in_specs=[pl.BlockSpec((2, _RB, _H), lambda i: (0, i, 0)),
                  _rows(_H), _rows(1), _bcast(1, _H)],
        out_specs=_rows(_H),
        out_shape=jax.ShapeDtypeStruct((_N, _H), jnp.float32),
    )(p, a, dinv, b)


def _stats(c):
    s = jax.ShapeDtypeStruct((1, _H), jnp.float32)
    return pl.pallas_call(_stats_body, out_shape=[s, s])(c)


def _norm(c, m, v, gm, bt, res, w, dinv):
    has_res = res is not None
    extra = [_rows(_H)] if has_res else []
    args = [c, m, v, gm, bt] + ([res] if has_res else []) + [w, dinv]
    s = jax.ShapeDtypeStruct((_N, _H), jnp.float32)
    return pl.pallas_call(
        functools.partial(_norm_body, has_res),
        grid=(_NBLK,),
        in_specs=[_rows(_H), _bcast(1, _H), _bcast(1, _H), _bcast(1, _H),
                  _bcast(1, _H)] + extra + [_bcast(_H, _H), _rows(1)],
        out_specs=[_rows(_H), _rows(_H)],
        out_shape=[s, s],
    )(*args)


def kernel(x, edge_index, batch, W1, b1, g1, be1, W2, b2, g2, be2,
           W3, b3, g3, be3, Wfc, bfc):
    f32 = jnp.float32
    pad = jnp.full((_EP - _E,), _N, jnp.int32)
    srcp = jnp.concatenate([edge_index[0], pad])
    dstp = jnp.concatenate([edge_index[1], pad])
    zrow_h = jnp.zeros((_RPT, _H), f32)
    zrow_d = jnp.zeros((_RPT, _DW), f32)
    ones_d = jnp.ones((_K, _DW), f32)

    degp = _deg_pass(dstp, ones_d, zrow_d)
    dp = degp[:, :_N, 0:1]

    dinv, a1 = pl.pallas_call(
        _prelude_body,
        grid=(_NBLK,),
        in_specs=[_rows(_D), _bcast(_D, _H),
                  pl.BlockSpec((2, _RB, 1), lambda i: (0, i, 0))],
        out_specs=[_rows(1), _rows(_H)],
        out_shape=[jax.ShapeDtypeStruct((_N, 1), f32),
                   jax.ShapeDtypeStruct((_N, _H), f32)],
    )(x, W1, dp)

    p1 = _seg_pass(_pad_rows(a1), srcp, dstp, zrow_h)
    c1 = _conv(p1[:, :_N], a1, dinv, b1.reshape(1, _H))
    m1, v1 = _stats(c1)
    h1, a2 = _norm(c1, m1, v1, g1.reshape(1, _H), be1.reshape(1, _H),
                   None, W2, dinv)

    p2 = _seg_pass(_pad_rows(a2), srcp, dstp, zrow_h)
    c2 = _conv(p2[:, :_N], a2, dinv, b2.reshape(1, _H))
    m2, v2 = _stats(c2)
    h2, a3 = _norm(c2, m2, v2, g2.reshape(1, _H), be2.reshape(1, _H),
                   h1, W3, dinv)

    p3 = _seg_pass(_pad_rows(a3), srcp, dstp, zrow_h)
    c3 = _conv(p3[:, :_N], a3, dinv, b3.reshape(1, _H))
    m3, v3 = _stats(c3)
    out = pl.pallas_call(
        _pool_body,
        out_shape=jax.ShapeDtypeStruct((_G, 1), f32),
    )(c3, m3, v3, g3.reshape(1, _H), be3.reshape(1, _H),
      batch.reshape(1, _N), Wfc, bfc.reshape(1, 1))
    return out.reshape(_G)


# speedup vs baseline: 13.0358x; 1.1074x over previous
import functools

import jax
import jax.numpy as jnp
from jax import lax
from jax.experimental import pallas as pl
from jax.experimental.pallas import tpu as pltpu
from jax.experimental.pallas import tpu_sc as plsc

_N = 10000
_E = 320000
_D = 128
_H = 64
_G = 64

_NW = 32
_K = 128
_EP = 327680
_CHUNKS = _EP // (_NW * _K)
_ROWS = _EP // _K
_NP = 10112
_RPT = _NP // 16
_DW = 8

_mesh = plsc.VectorSubcoreMesh(core_axis_name="c", subcore_axis_name="s")


@functools.partial(
    pl.kernel,
    mesh=_mesh,
    compiler_params=pltpu.CompilerParams(use_tc_tiling_on_sc=False),
    out_type=jax.ShapeDtypeStruct((2, _NP, _DW), jnp.float32),
    scratch_types=[
        pltpu.VMEM((_CHUNKS, _K), jnp.int32),
        pltpu.VMEM((_K, _DW), jnp.float32),
        pltpu.VMEM_SHARED((_NP, _DW), jnp.float32),
    ],
)
def _deg_pass(dst_hbm, ones_hbm, zrow_hbm, out_hbm, didx_v, ones_v, deg_sh):
    c = lax.axis_index("c")
    s = lax.axis_index("s")
    w = c * 16 + s
    pltpu.sync_copy(zrow_hbm, deg_sh.at[pl.ds(s * _RPT, _RPT)])
    pltpu.sync_copy(ones_hbm, ones_v)
    pltpu.sync_copy(dst_hbm.at[pl.ds(w * _CHUNKS, _CHUNKS)], didx_v)
    plsc.subcore_barrier()

    def body(i, carry):
        pltpu.sync_copy(ones_v, deg_sh.at[didx_v.at[i]], add=True)
        return carry

    lax.fori_loop(0, _CHUNKS, body, 0)
    plsc.subcore_barrier()
    pltpu.sync_copy(deg_sh.at[pl.ds(s * _RPT, _RPT)],
                    out_hbm.at[c, pl.ds(s * _RPT, _RPT)])


@functools.partial(
    pl.kernel,
    mesh=_mesh,
    compiler_params=pltpu.CompilerParams(use_tc_tiling_on_sc=False),
    out_type=jax.ShapeDtypeStruct((2, _NP, _H), jnp.float32),
    scratch_types=[
        pltpu.VMEM((_CHUNKS, _K), jnp.int32),
        pltpu.VMEM((_CHUNKS, _K), jnp.int32),
        pltpu.VMEM((_K, _H), jnp.float32),
        pltpu.VMEM((_K, _H), jnp.float32),
        pltpu.VMEM_SHARED((_NP, _H), jnp.float32),
        pltpu.SemaphoreType.DMA,
        pltpu.SemaphoreType.DMA,
    ],
)
def _seg_pass(a_hbm, src_hbm, dst_hbm, zrow_hbm, out_hbm,
              sidx_v, didx_v, buf0, buf1, agg_sh, sem0, sem1):
    c = lax.axis_index("c")
    s = lax.axis_index("s")
    w = c * 16 + s
    pltpu.sync_copy(zrow_hbm, agg_sh.at[pl.ds(s * _RPT, _RPT)])
    pltpu.sync_copy(src_hbm.at[pl.ds(w * _CHUNKS, _CHUNKS)], sidx_v)
    pltpu.sync_copy(dst_hbm.at[pl.ds(w * _CHUNKS, _CHUNKS)], didx_v)
    plsc.subcore_barrier()
    g0 = pltpu.async_copy(a_hbm.at[sidx_v.at[0]], buf0, sem0)

    def body(j, carry):
        i0 = 2 * j
        g1 = pltpu.async_copy(a_hbm.at[sidx_v.at[i0 + 1]], buf1, sem1)
        pltpu.make_async_copy(a_hbm.at[pl.ds(0, _K)], buf0, sem0).wait()
        pltpu.sync_copy(buf0, agg_sh.at[didx_v.at[i0]], add=True)

        @pl.when(j < _CHUNKS // 2 - 1)
        def _():
            pltpu.async_copy(a_hbm.at[sidx_v.at[i0 + 2]], buf0, sem0)

        g1.wait()
        pltpu.sync_copy(buf1, agg_sh.at[didx_v.at[i0 + 1]], add=True)
        return carry

    lax.fori_loop(0, _CHUNKS // 2, body, 0)
    plsc.subcore_barrier()
    pltpu.sync_copy(agg_sh.at[pl.ds(s * _RPT, _RPT)],
                    out_hbm.at[c, pl.ds(s * _RPT, _RPT)])


_RB = 2000
_NBLK = _N // _RB


def _prelude_body(x_ref, w_ref, dp_ref, dinv_ref, a_ref):
    deg = dp_ref[0] + dp_ref[1] + 1.0
    dinv = lax.rsqrt(deg)
    dinv_ref[...] = dinv
    xw = jnp.dot(x_ref[...], w_ref[...], preferred_element_type=jnp.float32,
                 precision=lax.Precision.HIGHEST)
    a_ref[...] = dinv * xw


def _conv_body(p_ref, a_ref, dinv_ref, b_ref, conv_ref):
    conv_ref[...] = (dinv_ref[...] * (p_ref[0] + p_ref[1] + a_ref[...])
                     + b_ref[...])


def _stats_body(c_ref, m_ref, v_ref):
    c = c_ref[...]
    m = jnp.mean(c, axis=0, keepdims=True)
    m_ref[...] = m
    v_ref[...] = jnp.mean((c - m) * (c - m), axis=0, keepdims=True)


def _norm_body(has_res, c_ref, m_ref, v_ref, gm_ref, bt_ref, *rest):
    if has_res:
        res_ref, w_ref, dinv_ref, h_ref, an_ref = rest
    else:
        w_ref, dinv_ref, h_ref, an_ref = rest
    h = ((c_ref[...] - m_ref[...]) / jnp.sqrt(v_ref[...] + 1e-5)
         * gm_ref[...] + bt_ref[...])
    if has_res:
        h = h + res_ref[...]
    h = jnp.maximum(h, 0.0)
    h_ref[...] = h
    hw = jnp.dot(h, w_ref[...], preferred_element_type=jnp.float32,
                 precision=lax.Precision.HIGHEST)
    an_ref[...] = dinv_ref[...] * hw


def _pool_body(c_ref, m_ref, v_ref, gm_ref, bt_ref, batch_ref,
               wfc_ref, bfc_ref, out_ref):
    h = ((c_ref[...] - m_ref[...]) / jnp.sqrt(v_ref[...] + 1e-5)
         * gm_ref[...] + bt_ref[...])
    h = jnp.maximum(h, 0.0)
    gids = lax.broadcasted_iota(jnp.int32, (_G, _N), 0)
    onehot = (gids == batch_ref[...]).astype(jnp.float32)
    counts = jnp.sum(onehot, axis=1, keepdims=True)
    pooled = jnp.dot(onehot, h, preferred_element_type=jnp.float32,
                     precision=lax.Precision.HIGHEST)
    pooled = pooled / jnp.maximum(counts, 1.0)
    out_ref[...] = jnp.dot(pooled, wfc_ref[...],
                           preferred_element_type=jnp.float32,
                           precision=lax.Precision.HIGHEST) + bfc_ref[...]


def _rows(bs):
    return pl.BlockSpec((_RB, bs), lambda i: (i, 0))


def _bcast(r, c):
    return pl.BlockSpec((r, c), lambda i: (0, 0))


def _pad_rows(a):
    return jnp.concatenate([a, jnp.zeros((_NP - _N, _H), jnp.float32)], axis=0)


def _conv(p, a, dinv, b):
    return pl.pallas_call(
        _conv_body,
        grid=(_NBLK,),
        in_specs=[pl.BlockSpec((2, _RB, _H), lambda i: (0, i, 0)),
                  _rows(_H), _rows(1), _bcast(1, _H)],
        out_specs=_rows(_H),
        out_shape=jax.ShapeDtypeStruct((_N, _H), jnp.float32),
    )(p, a, dinv, b)


def _stats(c):
    s = jax.ShapeDtypeStruct((1, _H), jnp.float32)
    return pl.pallas_call(_stats_body, out_shape=[s, s])(c)


def _norm(c, m, v, gm, bt, res, w, dinv):
    has_res = res is not None
    extra = [_rows(_H)] if has_res else []
    args = [c, m, v, gm, bt] + ([res] if has_res else []) + [w, dinv]
    s = jax.ShapeDtypeStruct((_N, _H), jnp.float32)
    return pl.pallas_call(
        functools.partial(_norm_body, has_res),
        grid=(_NBLK,),
        in_specs=[_rows(_H), _bcast(1, _H), _bcast(1, _H), _bcast(1, _H),
                  _bcast(1, _H)] + extra + [_bcast(_H, _H), _rows(1)],
        out_specs=[_rows(_H), _rows(_H)],
        out_shape=[s, s],
    )(*args)


def kernel(x, edge_index, batch, W1, b1, g1, be1, W2, b2, g2, be2,
           W3, b3, g3, be3, Wfc, bfc):
    f32 = jnp.float32
    pad = jnp.full((_EP - _E,), _N, jnp.int32)
    srcp = jnp.concatenate([edge_index[0], pad]).reshape(_ROWS, _K)
    dstp = jnp.concatenate([edge_index[1], pad]).reshape(_ROWS, _K)
    zrow_h = jnp.zeros((_RPT, _H), f32)
    zrow_d = jnp.zeros((_RPT, _DW), f32)
    ones_d = jnp.ones((_K, _DW), f32)

    degp = _deg_pass(dstp, ones_d, zrow_d)
    dp = degp[:, :_N, 0:1]

    dinv, a1 = pl.pallas_call(
        _prelude_body,
        grid=(_NBLK,),
        in_specs=[_rows(_D), _bcast(_D, _H),
                  pl.BlockSpec((2, _RB, 1), lambda i: (0, i, 0))],
        out_specs=[_rows(1), _rows(_H)],
        out_shape=[jax.ShapeDtypeStruct((_N, 1), f32),
                   jax.ShapeDtypeStruct((_N, _H), f32)],
    )(x, W1, dp)

    p1 = _seg_pass(_pad_rows(a1), srcp, dstp, zrow_h)
    c1 = _conv(p1[:, :_N], a1, dinv, b1.reshape(1, _H))
    m1, v1 = _stats(c1)
    h1, a2 = _norm(c1, m1, v1, g1.reshape(1, _H), be1.reshape(1, _H),
                   None, W2, dinv)

    p2 = _seg_pass(_pad_rows(a2), srcp, dstp, zrow_h)
    c2 = _conv(p2[:, :_N], a2, dinv, b2.reshape(1, _H))
    m2, v2 = _stats(c2)
    h2, a3 = _norm(c2, m2, v2, g2.reshape(1, _H), be2.reshape(1, _H),
                   h1, W3, dinv)

    p3 = _seg_pass(_pad_rows(a3), srcp, dstp, zrow_h)
    c3 = _conv(p3[:, :_N], a3, dinv, b3.reshape(1, _H))
    m3, v3 = _stats(c3)
    out = pl.pallas_call(
        _pool_body,
        out_shape=jax.ShapeDtypeStruct((_G, 1), f32),
    )(c3, m3, v3, g3.reshape(1, _H), be3.reshape(1, _H),
      batch.reshape(1, _N), Wfc, bfc.reshape(1, 1))
    return out.reshape(_G)


# speedup vs baseline: 26.3980x; 2.0250x over previous
import functools

import jax
import jax.numpy as jnp
from jax import lax
from jax.experimental import pallas as pl
from jax.experimental.pallas import tpu as pltpu
from jax.experimental.pallas import tpu_sc as plsc

_N = 10000
_E = 320000
_D = 128
_H = 64
_G = 64

_NW = 32
_K = 128
_EP = 327680
_CHUNKS = _EP // (_NW * _K)
_ROWS = _EP // _K
_NP = 10112
_RPT = _NP // 16
_DW = 8

_mesh = plsc.VectorSubcoreMesh(core_axis_name="c", subcore_axis_name="s")


@functools.partial(
    pl.kernel,
    mesh=_mesh,
    compiler_params=pltpu.CompilerParams(use_tc_tiling_on_sc=False),
    out_type=jax.ShapeDtypeStruct((2, _NP, _DW), jnp.float32),
    scratch_types=[
        pltpu.VMEM((_CHUNKS, _K), jnp.int32),
        pltpu.VMEM((_K, _DW), jnp.float32),
        pltpu.VMEM_SHARED((_NP, _DW), jnp.float32),
    ],
)
def _deg_pass(dst_hbm, ones_hbm, zrow_hbm, out_hbm, didx_v, ones_v, deg_sh):
    c = lax.axis_index("c")
    s = lax.axis_index("s")
    w = c * 16 + s
    pltpu.sync_copy(zrow_hbm, deg_sh.at[pl.ds(s * _RPT, _RPT)])
    pltpu.sync_copy(ones_hbm, ones_v)
    pltpu.sync_copy(dst_hbm.at[pl.ds(w * _CHUNKS, _CHUNKS)], didx_v)
    plsc.subcore_barrier()

    def body(i, carry):
        pltpu.sync_copy(ones_v, deg_sh.at[didx_v.at[i]], add=True)
        return carry

    lax.fori_loop(0, _CHUNKS, body, 0)
    plsc.subcore_barrier()
    pltpu.sync_copy(deg_sh.at[pl.ds(s * _RPT, _RPT)],
                    out_hbm.at[c, pl.ds(s * _RPT, _RPT)])


@functools.partial(
    pl.kernel,
    mesh=_mesh,
    compiler_params=pltpu.CompilerParams(use_tc_tiling_on_sc=False),
    out_type=jax.ShapeDtypeStruct((2, _NP, _H), jnp.float32),
    scratch_types=[
        pltpu.VMEM((_CHUNKS, _K), jnp.int32),
        pltpu.VMEM((_CHUNKS, _K), jnp.int32),
        pltpu.VMEM((_K, _H), jnp.float32),
        pltpu.VMEM((_K, _H), jnp.float32),
        pltpu.VMEM_SHARED((_NP, _H), jnp.float32),
        pltpu.VMEM_SHARED((_NP, _H), jnp.float32),
        pltpu.SemaphoreType.DMA,
        pltpu.SemaphoreType.DMA,
    ],
)
def _seg_pass(a_hbm, src_hbm, dst_hbm, zrow_hbm, out_hbm,
              sidx_v, didx_v, buf0, buf1, agg_sh, feat_sh, sem0, sem1):
    c = lax.axis_index("c")
    s = lax.axis_index("s")
    w = c * 16 + s
    pltpu.sync_copy(zrow_hbm, agg_sh.at[pl.ds(s * _RPT, _RPT)])
    pltpu.sync_copy(a_hbm.at[pl.ds(s * _RPT, _RPT)],
                    feat_sh.at[pl.ds(s * _RPT, _RPT)])
    pltpu.sync_copy(src_hbm.at[pl.ds(w * _CHUNKS, _CHUNKS)], sidx_v)
    pltpu.sync_copy(dst_hbm.at[pl.ds(w * _CHUNKS, _CHUNKS)], didx_v)
    plsc.subcore_barrier()
    g0 = pltpu.async_copy(feat_sh.at[sidx_v.at[0]], buf0, sem0)

    def body(j, carry):
        i0 = 2 * j
        g1 = pltpu.async_copy(feat_sh.at[sidx_v.at[i0 + 1]], buf1, sem1)
        pltpu.make_async_copy(feat_sh.at[pl.ds(0, _K)], buf0, sem0).wait()
        pltpu.sync_copy(buf0, agg_sh.at[didx_v.at[i0]], add=True)

        @pl.when(j < _CHUNKS // 2 - 1)
        def _():
            pltpu.async_copy(feat_sh.at[sidx_v.at[i0 + 2]], buf0, sem0)

        g1.wait()
        pltpu.sync_copy(buf1, agg_sh.at[didx_v.at[i0 + 1]], add=True)
        return carry

    lax.fori_loop(0, _CHUNKS // 2, body, 0)
    plsc.subcore_barrier()
    pltpu.sync_copy(agg_sh.at[pl.ds(s * _RPT, _RPT)],
                    out_hbm.at[c, pl.ds(s * _RPT, _RPT)])


_RB = 2000
_NBLK = _N // _RB


def _prelude_body(x_ref, w_ref, dp_ref, dinv_ref, a_ref):
    deg = dp_ref[0] + dp_ref[1] + 1.0
    dinv = lax.rsqrt(deg)
    dinv_ref[...] = dinv
    xw = jnp.dot(x_ref[...], w_ref[...], preferred_element_type=jnp.float32,
                 precision=lax.Precision.HIGHEST)
    a_ref[...] = dinv * xw


def _conv_body(p_ref, a_ref, dinv_ref, b_ref, conv_ref):
    conv_ref[...] = (dinv_ref[...] * (p_ref[0] + p_ref[1] + a_ref[...])
                     + b_ref[...])


def _stats_body(c_ref, m_ref, v_ref):
    c = c_ref[...]
    m = jnp.mean(c, axis=0, keepdims=True)
    m_ref[...] = m
    v_ref[...] = jnp.mean((c - m) * (c - m), axis=0, keepdims=True)


def _norm_body(has_res, c_ref, m_ref, v_ref, gm_ref, bt_ref, *rest):
    if has_res:
        res_ref, w_ref, dinv_ref, h_ref, an_ref = rest
    else:
        w_ref, dinv_ref, h_ref, an_ref = rest
    h = ((c_ref[...] - m_ref[...]) / jnp.sqrt(v_ref[...] + 1e-5)
         * gm_ref[...] + bt_ref[...])
    if has_res:
        h = h + res_ref[...]
    h = jnp.maximum(h, 0.0)
    h_ref[...] = h
    hw = jnp.dot(h, w_ref[...], preferred_element_type=jnp.float32,
                 precision=lax.Precision.HIGHEST)
    an_ref[...] = dinv_ref[...] * hw


def _pool_body(c_ref, m_ref, v_ref, gm_ref, bt_ref, batch_ref,
               wfc_ref, bfc_ref, out_ref):
    h = ((c_ref[...] - m_ref[...]) / jnp.sqrt(v_ref[...] + 1e-5)
         * gm_ref[...] + bt_ref[...])
    h = jnp.maximum(h, 0.0)
    gids = lax.broadcasted_iota(jnp.int32, (_G, _N), 0)
    onehot = (gids == batch_ref[...]).astype(jnp.float32)
    counts = jnp.sum(onehot, axis=1, keepdims=True)
    pooled = jnp.dot(onehot, h, preferred_element_type=jnp.float32,
                     precision=lax.Precision.HIGHEST)
    pooled = pooled / jnp.maximum(counts, 1.0)
    out_ref[...] = jnp.dot(pooled, wfc_ref[...],
                           preferred_element_type=jnp.float32,
                           precision=lax.Precision.HIGHEST) + bfc_ref[...]


def _rows(bs):
    return pl.BlockSpec((_RB, bs), lambda i: (i, 0))


def _bcast(r, c):
    return pl.BlockSpec((r, c), lambda i: (0, 0))


def _pad_rows(a):
    return jnp.concatenate([a, jnp.zeros((_NP - _N, _H), jnp.float32)], axis=0)


def _conv(p, a, dinv, b):
    return pl.pallas_call(
        _conv_body,
        grid=(_NBLK,),
        in_specs=[pl.BlockSpec((2, _RB, _H), lambda i: (0, i, 0)),
                  _rows(_H), _rows(1), _bcast(1, _H)],
        out_specs=_rows(_H),
        out_shape=jax.ShapeDtypeStruct((_N, _H), jnp.float32),
    )(p, a, dinv, b)


def _stats(c):
    s = jax.ShapeDtypeStruct((1, _H), jnp.float32)
    return pl.pallas_call(_stats_body, out_shape=[s, s])(c)


def _norm(c, m, v, gm, bt, res, w, dinv):
    has_res = res is not None
    extra = [_rows(_H)] if has_res else []
    args = [c, m, v, gm, bt] + ([res] if has_res else []) + [w, dinv]
    s = jax.ShapeDtypeStruct((_N, _H), jnp.float32)
    return pl.pallas_call(
        functools.partial(_norm_body, has_res),
        grid=(_NBLK,),
        in_specs=[_rows(_H), _bcast(1, _H), _bcast(1, _H), _bcast(1, _H),
                  _bcast(1, _H)] + extra + [_bcast(_H, _H), _rows(1)],
        out_specs=[_rows(_H), _rows(_H)],
        out_shape=[s, s],
    )(*args)


def kernel(x, edge_index, batch, W1, b1, g1, be1, W2, b2, g2, be2,
           W3, b3, g3, be3, Wfc, bfc):
    f32 = jnp.float32
    pad = jnp.full((_EP - _E,), _N, jnp.int32)
    srcp = jnp.concatenate([edge_index[0], pad]).reshape(_ROWS, _K)
    dstp = jnp.concatenate([edge_index[1], pad]).reshape(_ROWS, _K)
    zrow_h = jnp.zeros((_RPT, _H), f32)
    zrow_d = jnp.zeros((_RPT, _DW), f32)
    ones_d = jnp.ones((_K, _DW), f32)

    degp = _deg_pass(dstp, ones_d, zrow_d)
    dp = degp[:, :_N, 0:1]

    dinv, a1 = pl.pallas_call(
        _prelude_body,
        grid=(_NBLK,),
        in_specs=[_rows(_D), _bcast(_D, _H),
                  pl.BlockSpec((2, _RB, 1), lambda i: (0, i, 0))],
        out_specs=[_rows(1), _rows(_H)],
        out_shape=[jax.ShapeDtypeStruct((_N, 1), f32),
                   jax.ShapeDtypeStruct((_N, _H), f32)],
    )(x, W1, dp)

    p1 = _seg_pass(_pad_rows(a1), srcp, dstp, zrow_h)
    c1 = _conv(p1[:, :_N], a1, dinv, b1.reshape(1, _H))
    m1, v1 = _stats(c1)
    h1, a2 = _norm(c1, m1, v1, g1.reshape(1, _H), be1.reshape(1, _H),
                   None, W2, dinv)

    p2 = _seg_pass(_pad_rows(a2), srcp, dstp, zrow_h)
    c2 = _conv(p2[:, :_N], a2, dinv, b2.reshape(1, _H))
    m2, v2 = _stats(c2)
    h2, a3 = _norm(c2, m2, v2, g2.reshape(1, _H), be2.reshape(1, _H),
                   h1, W3, dinv)

    p3 = _seg_pass(_pad_rows(a3), srcp, dstp, zrow_h)
    c3 = _conv(p3[:, :_N], a3, dinv, b3.reshape(1, _H))
    m3, v3 = _stats(c3)
    out = pl.pallas_call(
        _pool_body,
        out_shape=jax.ShapeDtypeStruct((_G, 1), f32),
    )(c3, m3, v3, g3.reshape(1, _H), be3.reshape(1, _H),
      batch.reshape(1, _N), Wfc, bfc.reshape(1, 1))
    return out.reshape(_G)


# speedup vs baseline: 27.1688x; 1.0292x over previous
import functools

import jax
import jax.numpy as jnp
from jax import lax
from jax.experimental import pallas as pl
from jax.experimental.pallas import tpu as pltpu
from jax.experimental.pallas import tpu_sc as plsc

_N = 10000
_E = 320000
_D = 128
_H = 64
_G = 64

_NW = 32
_K = 128
_EP = 327680
_CHUNKS = _EP // (_NW * _K)
_ROWS = _EP // _K
_NP = 10112
_RPT = _NP // 16
_DW = 8

_mesh = plsc.VectorSubcoreMesh(core_axis_name="c", subcore_axis_name="s")


@functools.partial(
    pl.kernel,
    mesh=_mesh,
    compiler_params=pltpu.CompilerParams(use_tc_tiling_on_sc=False),
    out_type=jax.ShapeDtypeStruct((2, _NP, _DW), jnp.float32),
    scratch_types=[
        pltpu.VMEM((_CHUNKS, _K), jnp.int32),
        pltpu.VMEM((_K, _DW), jnp.float32),
        pltpu.VMEM_SHARED((_NP, _DW), jnp.float32),
    ],
)
def _deg_pass(dst_hbm, ones_hbm, zrow_hbm, out_hbm, didx_v, ones_v, deg_sh):
    c = lax.axis_index("c")
    s = lax.axis_index("s")
    w = c * 16 + s
    pltpu.sync_copy(zrow_hbm, deg_sh.at[pl.ds(s * _RPT, _RPT)])
    pltpu.sync_copy(ones_hbm, ones_v)
    pltpu.sync_copy(dst_hbm.at[pl.ds(w * _CHUNKS, _CHUNKS)], didx_v)
    plsc.subcore_barrier()

    def body(i, carry):
        pltpu.sync_copy(ones_v, deg_sh.at[didx_v.at[i]], add=True)
        return carry

    lax.fori_loop(0, _CHUNKS, body, 0)
    plsc.subcore_barrier()
    pltpu.sync_copy(deg_sh.at[pl.ds(s * _RPT, _RPT)],
                    out_hbm.at[c, pl.ds(s * _RPT, _RPT)])


@functools.partial(
    pl.kernel,
    mesh=_mesh,
    compiler_params=pltpu.CompilerParams(use_tc_tiling_on_sc=False),
    out_type=jax.ShapeDtypeStruct((2, _NP, _H), jnp.float32),
    scratch_types=[
        pltpu.VMEM((_CHUNKS, _K), jnp.int32),
        pltpu.VMEM((_CHUNKS, _K), jnp.int32),
        pltpu.VMEM((_K, _H), jnp.float32),
        pltpu.VMEM((_K, _H), jnp.float32),
        pltpu.VMEM_SHARED((_NP, _H), jnp.float32),
        pltpu.VMEM_SHARED((_NP, _H), jnp.float32),
        pltpu.SemaphoreType.DMA,
        pltpu.SemaphoreType.DMA,
    ],
)
def _seg_pass(a_hbm, src_hbm, dst_hbm, zrow_hbm, out_hbm,
              sidx_v, didx_v, buf0, buf1, agg_sh, feat_sh, sem0, sem1):
    c = lax.axis_index("c")
    s = lax.axis_index("s")
    w = c * 16 + s
    pltpu.sync_copy(zrow_hbm, agg_sh.at[pl.ds(s * _RPT, _RPT)])
    pltpu.sync_copy(a_hbm.at[pl.ds(s * _RPT, _RPT)],
                    feat_sh.at[pl.ds(s * _RPT, _RPT)])
    pltpu.sync_copy(src_hbm.at[pl.ds(w * _CHUNKS, _CHUNKS)], sidx_v)
    pltpu.sync_copy(dst_hbm.at[pl.ds(w * _CHUNKS, _CHUNKS)], didx_v)
    plsc.subcore_barrier()
    g0 = pltpu.async_copy(feat_sh.at[sidx_v.at[0]], buf0, sem0)

    def body(j, carry):
        i0 = 2 * j
        g1 = pltpu.async_copy(feat_sh.at[sidx_v.at[i0 + 1]], buf1, sem1)
        pltpu.make_async_copy(feat_sh.at[pl.ds(0, _K)], buf0, sem0).wait()
        pltpu.sync_copy(buf0, agg_sh.at[didx_v.at[i0]], add=True)

        @pl.when(j < _CHUNKS // 2 - 1)
        def _():
            pltpu.async_copy(feat_sh.at[sidx_v.at[i0 + 2]], buf0, sem0)

        g1.wait()
        pltpu.sync_copy(buf1, agg_sh.at[didx_v.at[i0 + 1]], add=True)
        return carry

    lax.fori_loop(0, _CHUNKS // 2, body, 0)
    plsc.subcore_barrier()
    pltpu.sync_copy(agg_sh.at[pl.ds(s * _RPT, _RPT)],
                    out_hbm.at[c, pl.ds(s * _RPT, _RPT)])


_RB = 1264
_NBLK = _NP // _RB


def _prelude_body(x_ref, w_ref, dp_ref, dinv_ref, a_ref):
    deg = dp_ref[0] + dp_ref[1] + 1.0
    dinv = lax.rsqrt(deg)
    dinv_ref[...] = dinv
    xw = jnp.dot(x_ref[...], w_ref[...], preferred_element_type=jnp.float32,
                 precision=lax.Precision.HIGHEST)
    a_ref[...] = dinv * xw


def _conv_body(p_ref, a_ref, dinv_ref, b_ref, conv_ref):
    conv_ref[...] = (dinv_ref[...] * (p_ref[0] + p_ref[1] + a_ref[...])
                     + b_ref[...])


def _stats_body(c_ref, m_ref, v_ref):
    c = c_ref[...]
    m = jnp.mean(c, axis=0, keepdims=True)
    m_ref[...] = m
    v_ref[...] = jnp.mean((c - m) * (c - m), axis=0, keepdims=True)


def _norm_body(has_res, c_ref, m_ref, v_ref, gm_ref, bt_ref, *rest):
    if has_res:
        res_ref, w_ref, dinv_ref, h_ref, an_ref = rest
    else:
        w_ref, dinv_ref, h_ref, an_ref = rest
    h = ((c_ref[...] - m_ref[...]) / jnp.sqrt(v_ref[...] + 1e-5)
         * gm_ref[...] + bt_ref[...])
    if has_res:
        h = h + res_ref[...]
    h = jnp.maximum(h, 0.0)
    h_ref[...] = h
    hw = jnp.dot(h, w_ref[...], preferred_element_type=jnp.float32,
                 precision=lax.Precision.HIGHEST)
    an_ref[...] = dinv_ref[...] * hw


def _pool_body(c_ref, m_ref, v_ref, gm_ref, bt_ref, batch_ref,
               wfc_ref, bfc_ref, out_ref):
    h = ((c_ref[...] - m_ref[...]) / jnp.sqrt(v_ref[...] + 1e-5)
         * gm_ref[...] + bt_ref[...])
    h = jnp.maximum(h, 0.0)
    gids = lax.broadcasted_iota(jnp.int32, (_G, _N), 0)
    onehot = (gids == batch_ref[...]).astype(jnp.float32)
    counts = jnp.sum(onehot, axis=1, keepdims=True)
    pooled = jnp.dot(onehot, h, preferred_element_type=jnp.float32,
                     precision=lax.Precision.HIGHEST)
    pooled = pooled / jnp.maximum(counts, 1.0)
    out_ref[...] = jnp.dot(pooled, wfc_ref[...],
                           preferred_element_type=jnp.float32,
                           precision=lax.Precision.HIGHEST) + bfc_ref[...]


def _rows(bs):
    return pl.BlockSpec((_RB, bs), lambda i: (i, 0))


def _bcast(r, c):
    return pl.BlockSpec((r, c), lambda i: (0, 0))


def _conv(p, a, dinv, b):
    return pl.pallas_call(
        _conv_body,
        grid=(_NBLK,),
        in_specs=[pl.BlockSpec((2, _RB, _H), lambda i: (0, i, 0)),
                  _rows(_H), _rows(1), _bcast(1, _H)],
        out_specs=_rows(_H),
        out_shape=jax.ShapeDtypeStruct((_N, _H), jnp.float32),
    )(p, a, dinv, b)


def _stats(c):
    s = jax.ShapeDtypeStruct((1, _H), jnp.float32)
    return pl.pallas_call(_stats_body, out_shape=[s, s])(c)


def _norm(c, m, v, gm, bt, res, w, dinv):
    has_res = res is not None
    extra = [_rows(_H)] if has_res else []
    args = [c, m, v, gm, bt] + ([res] if has_res else []) + [w, dinv]
    return pl.pallas_call(
        functools.partial(_norm_body, has_res),
        grid=(_NBLK,),
        in_specs=[_rows(_H), _bcast(1, _H), _bcast(1, _H), _bcast(1, _H),
                  _bcast(1, _H)] + extra + [_bcast(_H, _H), _rows(1)],
        out_specs=[_rows(_H), _rows(_H)],
        out_shape=[jax.ShapeDtypeStruct((_N, _H), jnp.float32),
                   jax.ShapeDtypeStruct((_NP, _H), jnp.float32)],
    )(*args)


def kernel(x, edge_index, batch, W1, b1, g1, be1, W2, b2, g2, be2,
           W3, b3, g3, be3, Wfc, bfc):
    f32 = jnp.float32
    pad = jnp.full((_EP - _E,), _N, jnp.int32)
    srcp = jnp.concatenate([edge_index[0], pad]).reshape(_ROWS, _K)
    dstp = jnp.concatenate([edge_index[1], pad]).reshape(_ROWS, _K)
    zrow_h = jnp.zeros((_RPT, _H), f32)
    zrow_d = jnp.zeros((_RPT, _DW), f32)
    ones_d = jnp.ones((_K, _DW), f32)

    degp = _deg_pass(dstp, ones_d, zrow_d)
    dp = degp[:, :_N, 0:1]

    dinv, a1 = pl.pallas_call(
        _prelude_body,
        grid=(_NBLK,),
        in_specs=[_rows(_D), _bcast(_D, _H),
                  pl.BlockSpec((2, _RB, 1), lambda i: (0, i, 0))],
        out_specs=[_rows(1), _rows(_H)],
        out_shape=[jax.ShapeDtypeStruct((_N, 1), f32),
                   jax.ShapeDtypeStruct((_NP, _H), f32)],
    )(x, W1, dp)

    p1 = _seg_pass(a1, srcp, dstp, zrow_h)
    c1 = _conv(p1, a1, dinv, b1.reshape(1, _H))
    m1, v1 = _stats(c1)
    h1, a2 = _norm(c1, m1, v1, g1.reshape(1, _H), be1.reshape(1, _H),
                   None, W2, dinv)

    p2 = _seg_pass(a2, srcp, dstp, zrow_h)
    c2 = _conv(p2, a2, dinv, b2.reshape(1, _H))
    m2, v2 = _stats(c2)
    h2, a3 = _norm(c2, m2, v2, g2.reshape(1, _H), be2.reshape(1, _H),
                   h1, W3, dinv)

    p3 = _seg_pass(a3, srcp, dstp, zrow_h)
    c3 = _conv(p3, a3, dinv, b3.reshape(1, _H))
    m3, v3 = _stats(c3)
    out = pl.pallas_call(
        _pool_body,
        out_shape=jax.ShapeDtypeStruct((_G, 1), f32),
    )(c3, m3, v3, g3.reshape(1, _H), be3.reshape(1, _H),
      batch.reshape(1, _N), Wfc, bfc.reshape(1, 1))
    return out.reshape(_G)


# speedup vs baseline: 28.6499x; 1.0545x over previous
import functools

import jax
import jax.numpy as jnp
from jax import lax
from jax.experimental import pallas as pl
from jax.experimental.pallas import tpu as pltpu
from jax.experimental.pallas import tpu_sc as plsc

_N = 10000
_E = 320000
_D = 128
_H = 64
_G = 64

_NW = 32
_K = 128
_EP = 327680
_CHUNKS = _EP // (_NW * _K)
_ROWS = _EP // _K
_NP = 10112
_RPT = _NP // 16
_DW = 8

_mesh = plsc.VectorSubcoreMesh(core_axis_name="c", subcore_axis_name="s")


@functools.partial(
    pl.kernel,
    mesh=_mesh,
    compiler_params=pltpu.CompilerParams(use_tc_tiling_on_sc=False),
    out_type=jax.ShapeDtypeStruct((2, _NP, _DW), jnp.float32),
    scratch_types=[
        pltpu.VMEM((_CHUNKS, _K), jnp.int32),
        pltpu.VMEM((_K, _DW), jnp.float32),
        pltpu.VMEM_SHARED((_NP, _DW), jnp.float32),
    ],
)
def _deg_pass(dst_hbm, ones_hbm, zrow_hbm, out_hbm, didx_v, ones_v, deg_sh):
    c = lax.axis_index("c")
    s = lax.axis_index("s")
    w = c * 16 + s
    pltpu.sync_copy(zrow_hbm, deg_sh.at[pl.ds(s * _RPT, _RPT)])
    pltpu.sync_copy(ones_hbm, ones_v)
    pltpu.sync_copy(dst_hbm.at[pl.ds(w * _CHUNKS, _CHUNKS)], didx_v)
    plsc.subcore_barrier()

    def body(i, carry):
        pltpu.sync_copy(ones_v, deg_sh.at[didx_v.at[i]], add=True)
        return carry

    lax.fori_loop(0, _CHUNKS, body, 0)
    plsc.subcore_barrier()
    pltpu.sync_copy(deg_sh.at[pl.ds(s * _RPT, _RPT)],
                    out_hbm.at[c, pl.ds(s * _RPT, _RPT)])


@functools.partial(
    pl.kernel,
    mesh=_mesh,
    compiler_params=pltpu.CompilerParams(use_tc_tiling_on_sc=False),
    out_type=jax.ShapeDtypeStruct((2, _NP, _H), jnp.float32),
    scratch_types=[
        pltpu.VMEM((_CHUNKS, _K), jnp.int32),
        pltpu.VMEM((_CHUNKS, _K), jnp.int32),
        pltpu.VMEM((_K, _H), jnp.float32),
        pltpu.VMEM((_K, _H), jnp.float32),
        pltpu.VMEM_SHARED((_NP, _H), jnp.float32),
        pltpu.VMEM_SHARED((_NP, _H), jnp.float32),
        pltpu.SemaphoreType.DMA,
        pltpu.SemaphoreType.DMA,
    ],
)
def _seg_pass(a_hbm, src_hbm, dst_hbm, zrow_hbm, out_hbm,
              sidx_v, didx_v, buf0, buf1, agg_sh, feat_sh, sem0, sem1):
    c = lax.axis_index("c")
    s = lax.axis_index("s")
    w = c * 16 + s
    pltpu.sync_copy(zrow_hbm, agg_sh.at[pl.ds(s * _RPT, _RPT)])
    pltpu.sync_copy(a_hbm.at[pl.ds(s * _RPT, _RPT)],
                    feat_sh.at[pl.ds(s * _RPT, _RPT)])
    pltpu.sync_copy(src_hbm.at[pl.ds(w * _CHUNKS, _CHUNKS)], sidx_v)
    pltpu.sync_copy(dst_hbm.at[pl.ds(w * _CHUNKS, _CHUNKS)], didx_v)
    plsc.subcore_barrier()
    g0 = pltpu.async_copy(feat_sh.at[sidx_v.at[0]], buf0, sem0)

    def body(j, carry):
        i0 = 2 * j
        g1 = pltpu.async_copy(feat_sh.at[sidx_v.at[i0 + 1]], buf1, sem1)
        pltpu.make_async_copy(feat_sh.at[pl.ds(0, _K)], buf0, sem0).wait()
        pltpu.sync_copy(buf0, agg_sh.at[didx_v.at[i0]], add=True)

        @pl.when(j < _CHUNKS // 2 - 1)
        def _():
            pltpu.async_copy(feat_sh.at[sidx_v.at[i0 + 2]], buf0, sem0)

        g1.wait()
        pltpu.sync_copy(buf1, agg_sh.at[didx_v.at[i0 + 1]], add=True)
        return carry

    lax.fori_loop(0, _CHUNKS // 2, body, 0)
    plsc.subcore_barrier()
    pltpu.sync_copy(agg_sh.at[pl.ds(s * _RPT, _RPT)],
                    out_hbm.at[c, pl.ds(s * _RPT, _RPT)])


_RB = 1264
_NBLK = _NP // _RB


def _xw_body(x_ref, w_ref, xw_ref):
    xw_ref[...] = jnp.dot(x_ref[...], w_ref[...],
                          preferred_element_type=jnp.float32,
                          precision=lax.Precision.HIGHEST)


def _scale_body(xw_ref, dp_ref, dinv_ref, a_ref):
    deg = dp_ref[0, :, 0:1] + dp_ref[1, :, 0:1] + 1.0
    dinv = lax.rsqrt(deg)
    dinv_ref[...] = dinv
    a_ref[...] = dinv * xw_ref[...]


def _conv_body(p_ref, a_ref, dinv_ref, b_ref, conv_ref, m_ref, v_ref):
    i = pl.program_id(0)
    c = (dinv_ref[...] * (p_ref[0] + p_ref[1] + a_ref[...]) + b_ref[...])
    conv_ref[...] = c
    rows = lax.broadcasted_iota(jnp.int32, (_RB, 1), 0) + i * _RB
    cm = jnp.where(rows < _N, c, 0.0)

    @pl.when(i == 0)
    def _():
        m_ref[...] = jnp.zeros_like(m_ref)
        v_ref[...] = jnp.zeros_like(v_ref)

    m_ref[...] += jnp.sum(cm, axis=0, keepdims=True)
    v_ref[...] += jnp.sum(cm * cm, axis=0, keepdims=True)

    @pl.when(i == _NBLK - 1)
    def _():
        m = m_ref[...] / _N
        m_ref[...] = m
        v_ref[...] = v_ref[...] / _N - m * m


def _norm_body(has_res, c_ref, m_ref, v_ref, gm_ref, bt_ref, *rest):
    if has_res:
        res_ref, w_ref, dinv_ref, h_ref, an_ref = rest
    else:
        w_ref, dinv_ref, h_ref, an_ref = rest
    h = ((c_ref[...] - m_ref[...]) / jnp.sqrt(v_ref[...] + 1e-5)
         * gm_ref[...] + bt_ref[...])
    if has_res:
        h = h + res_ref[...]
    h = jnp.maximum(h, 0.0)
    h_ref[...] = h
    hw = jnp.dot(h, w_ref[...], preferred_element_type=jnp.float32,
                 precision=lax.Precision.HIGHEST)
    an_ref[...] = dinv_ref[...] * hw


def _pool_body(c_ref, m_ref, v_ref, gm_ref, bt_ref, batch_ref,
               wfc_ref, bfc_ref, out_ref):
    h = ((c_ref[...] - m_ref[...]) / jnp.sqrt(v_ref[...] + 1e-5)
         * gm_ref[...] + bt_ref[...])
    h = jnp.maximum(h, 0.0)
    gids = lax.broadcasted_iota(jnp.int32, (_G, _N), 0)
    onehot = (gids == batch_ref[...]).astype(jnp.float32)
    counts = jnp.sum(onehot, axis=1, keepdims=True)
    pooled = jnp.dot(onehot, h, preferred_element_type=jnp.float32,
                     precision=lax.Precision.HIGHEST)
    pooled = pooled / jnp.maximum(counts, 1.0)
    out_ref[...] = jnp.dot(pooled, wfc_ref[...],
                           preferred_element_type=jnp.float32,
                           precision=lax.Precision.HIGHEST) + bfc_ref[...]


def _rows(bs):
    return pl.BlockSpec((_RB, bs), lambda i: (i, 0))


def _bcast(r, c):
    return pl.BlockSpec((r, c), lambda i: (0, 0))


def _conv(p, a, dinv, b):
    s = jax.ShapeDtypeStruct((1, _H), jnp.float32)
    return pl.pallas_call(
        _conv_body,
        grid=(_NBLK,),
        in_specs=[pl.BlockSpec((2, _RB, _H), lambda i: (0, i, 0)),
                  _rows(_H), _rows(1), _bcast(1, _H)],
        out_specs=[_rows(_H), _bcast(1, _H), _bcast(1, _H)],
        out_shape=[jax.ShapeDtypeStruct((_N, _H), jnp.float32), s, s],
    )(p, a, dinv, b)


def _norm(c, m, v, gm, bt, res, w, dinv):
    has_res = res is not None
    extra = [_rows(_H)] if has_res else []
    args = [c, m, v, gm, bt] + ([res] if has_res else []) + [w, dinv]
    return pl.pallas_call(
        functools.partial(_norm_body, has_res),
        grid=(_NBLK,),
        in_specs=[_rows(_H), _bcast(1, _H), _bcast(1, _H), _bcast(1, _H),
                  _bcast(1, _H)] + extra + [_bcast(_H, _H), _rows(1)],
        out_specs=[_rows(_H), _rows(_H)],
        out_shape=[jax.ShapeDtypeStruct((_N, _H), jnp.float32),
                   jax.ShapeDtypeStruct((_NP, _H), jnp.float32)],
    )(*args)


def kernel(x, edge_index, batch, W1, b1, g1, be1, W2, b2, g2, be2,
           W3, b3, g3, be3, Wfc, bfc):
    f32 = jnp.float32
    pad = jnp.full((_EP - _E,), _N, jnp.int32)
    srcp = jnp.concatenate([edge_index[0], pad]).reshape(_ROWS, _K)
    dstp = jnp.concatenate([edge_index[1], pad]).reshape(_ROWS, _K)
    zrow_h = jnp.zeros((_RPT, _H), f32)
    zrow_d = jnp.zeros((_RPT, _DW), f32)
    ones_d = jnp.ones((_K, _DW), f32)

    degp = _deg_pass(dstp, ones_d, zrow_d)
    xw = pl.pallas_call(
        _xw_body,
        grid=(_NBLK,),
        in_specs=[_rows(_D), _bcast(_D, _H)],
        out_specs=_rows(_H),
        out_shape=jax.ShapeDtypeStruct((_NP, _H), f32),
    )(x, W1)
    dinv, a1 = pl.pallas_call(
        _scale_body,
        grid=(_NBLK,),
        in_specs=[_rows(_H),
                  pl.BlockSpec((2, _RB, _DW), lambda i: (0, i, 0))],
        out_specs=[_rows(1), _rows(_H)],
        out_shape=[jax.ShapeDtypeStruct((_N, 1), f32),
                   jax.ShapeDtypeStruct((_NP, _H), f32)],
    )(xw, degp)

    p1 = _seg_pass(a1, srcp, dstp, zrow_h)
    c1, m1, v1 = _conv(p1, a1, dinv, b1.reshape(1, _H))
    h1, a2 = _norm(c1, m1, v1, g1.reshape(1, _H), be1.reshape(1, _H),
                   None, W2, dinv)

    p2 = _seg_pass(a2, srcp, dstp, zrow_h)
    c2, m2, v2 = _conv(p2, a2, dinv, b2.reshape(1, _H))
    h2, a3 = _norm(c2, m2, v2, g2.reshape(1, _H), be2.reshape(1, _H),
                   h1, W3, dinv)

    p3 = _seg_pass(a3, srcp, dstp, zrow_h)
    c3, m3, v3 = _conv(p3, a3, dinv, b3.reshape(1, _H))
    out = pl.pallas_call(
        _pool_body,
        out_shape=jax.ShapeDtypeStruct((_G, 1), f32),
    )(c3, m3, v3, g3.reshape(1, _H), be3.reshape(1, _H),
      batch.reshape(1, _N), Wfc, bfc.reshape(1, 1))
    return out.reshape(_G)


# speedup vs baseline: 29.4342x; 1.0274x over previous
import functools

import jax
import jax.numpy as jnp
from jax import lax
from jax.experimental import pallas as pl
from jax.experimental.pallas import tpu as pltpu
from jax.experimental.pallas import tpu_sc as plsc

_N = 10000
_E = 320000
_D = 128
_H = 64
_G = 64

_NW = 32
_K = 128
_EP = 327680
_CHUNKS = _EP // (_NW * _K)
_ROWS = _EP // _K
_NP = 10112
_RPT = _NP // 16
_DW = 8

_mesh = plsc.VectorSubcoreMesh(core_axis_name="c", subcore_axis_name="s")


@functools.partial(
    pl.kernel,
    mesh=_mesh,
    compiler_params=pltpu.CompilerParams(use_tc_tiling_on_sc=False),
    out_type=jax.ShapeDtypeStruct((2, _NP, _DW), jnp.float32),
    scratch_types=[
        pltpu.VMEM((_CHUNKS, _K), jnp.int32),
        pltpu.VMEM((_K, _DW), jnp.float32),
        pltpu.VMEM_SHARED((_NP, _DW), jnp.float32),
    ],
)
def _deg_pass(dst_hbm, ones_hbm, zrow_hbm, out_hbm, didx_v, ones_v, deg_sh):
    c = lax.axis_index("c")
    s = lax.axis_index("s")
    w = c * 16 + s
    pltpu.sync_copy(zrow_hbm, deg_sh.at[pl.ds(s * _RPT, _RPT)])
    pltpu.sync_copy(ones_hbm, ones_v)
    pltpu.sync_copy(dst_hbm.at[pl.ds(w * _CHUNKS, _CHUNKS)], didx_v)
    plsc.subcore_barrier()

    def body(i, carry):
        pltpu.sync_copy(ones_v, deg_sh.at[didx_v.at[i]], add=True)
        return carry

    lax.fori_loop(0, _CHUNKS, body, 0)
    plsc.subcore_barrier()
    pltpu.sync_copy(deg_sh.at[pl.ds(s * _RPT, _RPT)],
                    out_hbm.at[c, pl.ds(s * _RPT, _RPT)])


@functools.partial(
    pl.kernel,
    mesh=_mesh,
    compiler_params=pltpu.CompilerParams(use_tc_tiling_on_sc=False),
    out_type=jax.ShapeDtypeStruct((2, _NP, _H), jnp.float32),
    scratch_types=[
        pltpu.VMEM((_CHUNKS, _K), jnp.int32),
        pltpu.VMEM((_CHUNKS, _K), jnp.int32),
        pltpu.VMEM((_K, _H), jnp.float32),
        pltpu.VMEM((_K, _H), jnp.float32),
        pltpu.VMEM_SHARED((_NP, _H), jnp.float32),
        pltpu.VMEM_SHARED((_NP, _H), jnp.float32),
        pltpu.SemaphoreType.DMA,
        pltpu.SemaphoreType.DMA,
    ],
)
def _seg_pass(a_hbm, src_hbm, dst_hbm, zrow_hbm, out_hbm,
              sidx_v, didx_v, buf0, buf1, agg_sh, feat_sh, sem0, sem1):
    c = lax.axis_index("c")
    s = lax.axis_index("s")
    w = c * 16 + s
    pltpu.sync_copy(zrow_hbm, agg_sh.at[pl.ds(s * _RPT, _RPT)])
    pltpu.sync_copy(a_hbm.at[pl.ds(s * _RPT, _RPT)],
                    feat_sh.at[pl.ds(s * _RPT, _RPT)])
    pltpu.sync_copy(src_hbm.at[pl.ds(w * _CHUNKS, _CHUNKS)], sidx_v)
    pltpu.sync_copy(dst_hbm.at[pl.ds(w * _CHUNKS, _CHUNKS)], didx_v)
    plsc.subcore_barrier()
    g0 = pltpu.async_copy(feat_sh.at[sidx_v.at[0]], buf0, sem0)

    def body(j, carry):
        i0 = 2 * j
        g1 = pltpu.async_copy(feat_sh.at[sidx_v.at[i0 + 1]], buf1, sem1)
        pltpu.make_async_copy(feat_sh.at[pl.ds(0, _K)], buf0, sem0).wait()
        pltpu.sync_copy(buf0, agg_sh.at[didx_v.at[i0]], add=True)

        @pl.when(j < _CHUNKS // 2 - 1)
        def _():
            pltpu.async_copy(feat_sh.at[sidx_v.at[i0 + 2]], buf0, sem0)

        g1.wait()
        pltpu.sync_copy(buf1, agg_sh.at[didx_v.at[i0 + 1]], add=True)
        return carry

    lax.fori_loop(0, _CHUNKS // 2, body, 0)
    plsc.subcore_barrier()
    pltpu.sync_copy(agg_sh.at[pl.ds(s * _RPT, _RPT)],
                    out_hbm.at[c, pl.ds(s * _RPT, _RPT)])


_RB = 2528
_NBLK = _NP // _RB


def _xw_body(x_ref, w_ref, xw_ref):
    xw_ref[...] = jnp.dot(x_ref[...], w_ref[...],
                          preferred_element_type=jnp.float32,
                          precision=lax.Precision.HIGHEST)


def _scale_body(xw_ref, dp_ref, dinv_ref, a_ref):
    deg = dp_ref[0, :, 0:1] + dp_ref[1, :, 0:1] + 1.0
    dinv = lax.rsqrt(deg)
    dinv_ref[...] = dinv
    a_ref[...] = dinv * xw_ref[...]


def _conv_body(p_ref, a_ref, dinv_ref, b_ref, conv_ref, m_ref, v_ref):
    i = pl.program_id(0)
    c = (dinv_ref[...] * (p_ref[0] + p_ref[1] + a_ref[...]) + b_ref[...])
    conv_ref[...] = c
    rows = lax.broadcasted_iota(jnp.int32, (_RB, 1), 0) + i * _RB
    cm = jnp.where(rows < _N, c, 0.0)

    @pl.when(i == 0)
    def _():
        m_ref[...] = jnp.zeros_like(m_ref)
        v_ref[...] = jnp.zeros_like(v_ref)

    m_ref[...] += jnp.sum(cm, axis=0, keepdims=True)
    v_ref[...] += jnp.sum(cm * cm, axis=0, keepdims=True)

    @pl.when(i == _NBLK - 1)
    def _():
        m = m_ref[...] / _N
        m_ref[...] = m
        v_ref[...] = v_ref[...] / _N - m * m


def _norm_body(has_res, c_ref, m_ref, v_ref, gm_ref, bt_ref, *rest):
    if has_res:
        res_ref, w_ref, dinv_ref, h_ref, an_ref = rest
    else:
        w_ref, dinv_ref, h_ref, an_ref = rest
    h = ((c_ref[...] - m_ref[...]) / jnp.sqrt(v_ref[...] + 1e-5)
         * gm_ref[...] + bt_ref[...])
    if has_res:
        h = h + res_ref[...]
    h = jnp.maximum(h, 0.0)
    h_ref[...] = h
    hw = jnp.dot(h, w_ref[...], preferred_element_type=jnp.float32,
                 precision=lax.Precision.HIGHEST)
    an_ref[...] = dinv_ref[...] * hw


def _pool_body(c_ref, m_ref, v_ref, gm_ref, bt_ref, batch_ref,
               wfc_ref, bfc_ref, out_ref):
    h = ((c_ref[...] - m_ref[...]) / jnp.sqrt(v_ref[...] + 1e-5)
         * gm_ref[...] + bt_ref[...])
    h = jnp.maximum(h, 0.0)
    gids = lax.broadcasted_iota(jnp.int32, (_G, _N), 0)
    onehot = (gids == batch_ref[...]).astype(jnp.float32)
    counts = jnp.sum(onehot, axis=1, keepdims=True)
    pooled = jnp.dot(onehot, h, preferred_element_type=jnp.float32,
                     precision=lax.Precision.HIGHEST)
    pooled = pooled / jnp.maximum(counts, 1.0)
    out_ref[...] = jnp.dot(pooled, wfc_ref[...],
                           preferred_element_type=jnp.float32,
                           precision=lax.Precision.HIGHEST) + bfc_ref[...]


def _rows(bs):
    return pl.BlockSpec((_RB, bs), lambda i: (i, 0))


def _bcast(r, c):
    return pl.BlockSpec((r, c), lambda i: (0, 0))


def _conv(p, a, dinv, b):
    s = jax.ShapeDtypeStruct((1, _H), jnp.float32)
    return pl.pallas_call(
        _conv_body,
        grid=(_NBLK,),
        in_specs=[pl.BlockSpec((2, _RB, _H), lambda i: (0, i, 0)),
                  _rows(_H), _rows(1), _bcast(1, _H)],
        out_specs=[_rows(_H), _bcast(1, _H), _bcast(1, _H)],
        out_shape=[jax.ShapeDtypeStruct((_N, _H), jnp.float32), s, s],
    )(p, a, dinv, b)


def _norm(c, m, v, gm, bt, res, w, dinv):
    has_res = res is not None
    extra = [_rows(_H)] if has_res else []
    args = [c, m, v, gm, bt] + ([res] if has_res else []) + [w, dinv]
    return pl.pallas_call(
        functools.partial(_norm_body, has_res),
        grid=(_NBLK,),
        in_specs=[_rows(_H), _bcast(1, _H), _bcast(1, _H), _bcast(1, _H),
                  _bcast(1, _H)] + extra + [_bcast(_H, _H), _rows(1)],
        out_specs=[_rows(_H), _rows(_H)],
        out_shape=[jax.ShapeDtypeStruct((_N, _H), jnp.float32),
                   jax.ShapeDtypeStruct((_NP, _H), jnp.float32)],
    )(*args)


def kernel(x, edge_index, batch, W1, b1, g1, be1, W2, b2, g2, be2,
           W3, b3, g3, be3, Wfc, bfc):
    f32 = jnp.float32
    pad = jnp.full((_EP - _E,), _N, jnp.int32)
    srcp = jnp.concatenate([edge_index[0], pad]).reshape(_ROWS, _K)
    dstp = jnp.concatenate([edge_index[1], pad]).reshape(_ROWS, _K)
    zrow_h = jnp.zeros((_RPT, _H), f32)
    zrow_d = jnp.zeros((_RPT, _DW), f32)
    ones_d = jnp.ones((_K, _DW), f32)

    degp = _deg_pass(dstp, ones_d, zrow_d)
    xw = pl.pallas_call(
        _xw_body,
        grid=(_NBLK,),
        in_specs=[_rows(_D), _bcast(_D, _H)],
        out_specs=_rows(_H),
        out_shape=jax.ShapeDtypeStruct((_NP, _H), f32),
    )(x, W1)
    dinv, a1 = pl.pallas_call(
        _scale_body,
        grid=(_NBLK,),
        in_specs=[_rows(_H),
                  pl.BlockSpec((2, _RB, _DW), lambda i: (0, i, 0))],
        out_specs=[_rows(1), _rows(_H)],
        out_shape=[jax.ShapeDtypeStruct((_N, 1), f32),
                   jax.ShapeDtypeStruct((_NP, _H), f32)],
    )(xw, degp)

    p1 = _seg_pass(a1, srcp, dstp, zrow_h)
    c1, m1, v1 = _conv(p1, a1, dinv, b1.reshape(1, _H))
    h1, a2 = _norm(c1, m1, v1, g1.reshape(1, _H), be1.reshape(1, _H),
                   None, W2, dinv)

    p2 = _seg_pass(a2, srcp, dstp, zrow_h)
    c2, m2, v2 = _conv(p2, a2, dinv, b2.reshape(1, _H))
    h2, a3 = _norm(c2, m2, v2, g2.reshape(1, _H), be2.reshape(1, _H),
                   h1, W3, dinv)

    p3 = _seg_pass(a3, srcp, dstp, zrow_h)
    c3, m3, v3 = _conv(p3, a3, dinv, b3.reshape(1, _H))
    out = pl.pallas_call(
        _pool_body,
        out_shape=jax.ShapeDtypeStruct((_G, 1), f32),
    )(c3, m3, v3, g3.reshape(1, _H), be3.reshape(1, _H),
      batch.reshape(1, _N), Wfc, bfc.reshape(1, 1))
    return out.reshape(_G)


# speedup vs baseline: 29.7103x; 1.0094x over previous
import functools

import jax
import jax.numpy as jnp
from jax import lax
from jax.experimental import pallas as pl
from jax.experimental.pallas import tpu as pltpu
from jax.experimental.pallas import tpu_sc as plsc

_N = 10000
_E = 320000
_D = 128
_H = 64
_G = 64

_NW = 32
_K = 128
_ROWS = _E // _K
_CB = _ROWS // _NW
_CX = _ROWS - _CB * _NW
_NP = 10112
_RPT = _NP // 16
_DW = 8

_mesh = plsc.VectorSubcoreMesh(core_axis_name="c", subcore_axis_name="s")


@functools.partial(
    pl.kernel,
    mesh=_mesh,
    compiler_params=pltpu.CompilerParams(use_tc_tiling_on_sc=False),
    out_type=jax.ShapeDtypeStruct((2, _NP, _DW), jnp.float32),
    scratch_types=[
        pltpu.VMEM((_CB + 1, _K), jnp.int32),
        pltpu.VMEM((_K, _DW), jnp.float32),
        pltpu.VMEM_SHARED((_NP, _DW), jnp.float32),
    ],
)
def _deg_pass(dst_hbm, ones_hbm, zrow_hbm, out_hbm, didx_v, ones_v, deg_sh):
    c = lax.axis_index("c")
    s = lax.axis_index("s")
    w = c * 16 + s
    start = w * _CB + jnp.minimum(w, _CX)
    extra = w < _CX
    pltpu.sync_copy(zrow_hbm, deg_sh.at[pl.ds(s * _RPT, _RPT)])
    pltpu.sync_copy(ones_hbm, ones_v)
    pltpu.sync_copy(dst_hbm.at[pl.ds(start, _CB)], didx_v.at[pl.ds(0, _CB)])

    @pl.when(extra)
    def _():
        pltpu.sync_copy(dst_hbm.at[pl.ds(start + _CB, 1)],
                        didx_v.at[pl.ds(_CB, 1)])

    plsc.subcore_barrier()

    def body(i, carry):
        pltpu.sync_copy(ones_v, deg_sh.at[didx_v.at[i]], add=True)
        return carry

    lax.fori_loop(0, _CB + extra.astype(jnp.int32), body, 0)
    plsc.subcore_barrier()
    pltpu.sync_copy(deg_sh.at[pl.ds(s * _RPT, _RPT)],
                    out_hbm.at[c, pl.ds(s * _RPT, _RPT)])


@functools.partial(
    pl.kernel,
    mesh=_mesh,
    compiler_params=pltpu.CompilerParams(use_tc_tiling_on_sc=False),
    out_type=jax.ShapeDtypeStruct((2, _NP, _H), jnp.float32),
    scratch_types=[
        pltpu.VMEM((_CB + 1, _K), jnp.int32),
        pltpu.VMEM((_CB + 1, _K), jnp.int32),
        pltpu.VMEM((_K, _H), jnp.float32),
        pltpu.VMEM((_K, _H), jnp.float32),
        pltpu.VMEM_SHARED((_NP, _H), jnp.float32),
        pltpu.VMEM_SHARED((_NP, _H), jnp.float32),
        pltpu.SemaphoreType.DMA,
        pltpu.SemaphoreType.DMA,
    ],
)
def _seg_pass(a_hbm, src_hbm, dst_hbm, zrow_hbm, out_hbm,
              sidx_v, didx_v, buf0, buf1, agg_sh, feat_sh, sem0, sem1):
    c = lax.axis_index("c")
    s = lax.axis_index("s")
    w = c * 16 + s
    start = w * _CB + jnp.minimum(w, _CX)
    extra = w < _CX
    pltpu.sync_copy(zrow_hbm, agg_sh.at[pl.ds(s * _RPT, _RPT)])
    pltpu.sync_copy(a_hbm.at[pl.ds(s * _RPT, _RPT)],
                    feat_sh.at[pl.ds(s * _RPT, _RPT)])
    pltpu.sync_copy(src_hbm.at[pl.ds(start, _CB)], sidx_v.at[pl.ds(0, _CB)])
    pltpu.sync_copy(dst_hbm.at[pl.ds(start, _CB)], didx_v.at[pl.ds(0, _CB)])

    @pl.when(extra)
    def _():
        pltpu.sync_copy(src_hbm.at[pl.ds(start + _CB, 1)],
                        sidx_v.at[pl.ds(_CB, 1)])
        pltpu.sync_copy(dst_hbm.at[pl.ds(start + _CB, 1)],
                        didx_v.at[pl.ds(_CB, 1)])

    plsc.subcore_barrier()
    g0 = pltpu.async_copy(feat_sh.at[sidx_v.at[0]], buf0, sem0)

    def body(j, carry):
        i0 = 2 * j
        g1 = pltpu.async_copy(feat_sh.at[sidx_v.at[i0 + 1]], buf1, sem1)
        pltpu.make_async_copy(feat_sh.at[pl.ds(0, _K)], buf0, sem0).wait()
        pltpu.sync_copy(buf0, agg_sh.at[didx_v.at[i0]], add=True)

        @pl.when(j < _CB // 2 - 1)
        def _():
            pltpu.async_copy(feat_sh.at[sidx_v.at[i0 + 2]], buf0, sem0)

        g1.wait()
        pltpu.sync_copy(buf1, agg_sh.at[didx_v.at[i0 + 1]], add=True)
        return carry

    lax.fori_loop(0, _CB // 2, body, 0)

    @pl.when(extra)
    def _():
        pltpu.sync_copy(feat_sh.at[sidx_v.at[_CB]], buf0)
        pltpu.sync_copy(buf0, agg_sh.at[didx_v.at[_CB]], add=True)

    plsc.subcore_barrier()
    pltpu.sync_copy(agg_sh.at[pl.ds(s * _RPT, _RPT)],
                    out_hbm.at[c, pl.ds(s * _RPT, _RPT)])


_RB = 2528
_NBLK = _NP // _RB


def _xw_body(x_ref, w_ref, xw_ref):
    xw_ref[...] = jnp.dot(x_ref[...], w_ref[...],
                          preferred_element_type=jnp.float32,
                          precision=lax.Precision.HIGHEST)


def _scale_body(xw_ref, dp_ref, dinv_ref, a_ref):
    deg = dp_ref[0, :, 0:1] + dp_ref[1, :, 0:1] + 1.0
    dinv = lax.rsqrt(deg)
    dinv_ref[...] = dinv
    a_ref[...] = dinv * xw_ref[...]


def _conv_body(p_ref, a_ref, dinv_ref, b_ref, conv_ref, m_ref, v_ref):
    i = pl.program_id(0)
    c = (dinv_ref[...] * (p_ref[0] + p_ref[1] + a_ref[...]) + b_ref[...])
    conv_ref[...] = c
    rows = lax.broadcasted_iota(jnp.int32, (_RB, 1), 0) + i * _RB
    cm = jnp.where(rows < _N, c, 0.0)

    @pl.when(i == 0)
    def _():
        m_ref[...] = jnp.zeros_like(m_ref)
        v_ref[...] = jnp.zeros_like(v_ref)

    m_ref[...] += jnp.sum(cm, axis=0, keepdims=True)
    v_ref[...] += jnp.sum(cm * cm, axis=0, keepdims=True)

    @pl.when(i == _NBLK - 1)
    def _():
        m = m_ref[...] / _N
        m_ref[...] = m
        v_ref[...] = v_ref[...] / _N - m * m


def _norm_body(has_res, c_ref, m_ref, v_ref, gm_ref, bt_ref, *rest):
    if has_res:
        res_ref, w_ref, dinv_ref, h_ref, an_ref = rest
    else:
        w_ref, dinv_ref, h_ref, an_ref = rest
    h = ((c_ref[...] - m_ref[...]) / jnp.sqrt(v_ref[...] + 1e-5)
         * gm_ref[...] + bt_ref[...])
    if has_res:
        h = h + res_ref[...]
    h = jnp.maximum(h, 0.0)
    h_ref[...] = h
    hw = jnp.dot(h, w_ref[...], preferred_element_type=jnp.float32,
                 precision=lax.Precision.HIGHEST)
    an_ref[...] = dinv_ref[...] * hw


def _pool_body(c_ref, m_ref, v_ref, gm_ref, bt_ref, batch_ref,
               wfc_ref, bfc_ref, out_ref):
    h = ((c_ref[...] - m_ref[...]) / jnp.sqrt(v_ref[...] + 1e-5)
         * gm_ref[...] + bt_ref[...])
    h = jnp.maximum(h, 0.0)
    gids = lax.broadcasted_iota(jnp.int32, (_G, _N), 0)
    onehot = (gids == batch_ref[...]).astype(jnp.float32)
    counts = jnp.sum(onehot, axis=1, keepdims=True)
    pooled = jnp.dot(onehot, h, preferred_element_type=jnp.float32,
                     precision=lax.Precision.HIGHEST)
    pooled = pooled / jnp.maximum(counts, 1.0)
    out_ref[...] = jnp.dot(pooled, wfc_ref[...],
                           preferred_element_type=jnp.float32,
                           precision=lax.Precision.HIGHEST) + bfc_ref[...]


def _rows(bs):
    return pl.BlockSpec((_RB, bs), lambda i: (i, 0))


def _bcast(r, c):
    return pl.BlockSpec((r, c), lambda i: (0, 0))


def _conv(p, a, dinv, b):
    s = jax.ShapeDtypeStruct((1, _H), jnp.float32)
    return pl.pallas_call(
        _conv_body,
        grid=(_NBLK,),
        in_specs=[pl.BlockSpec((2, _RB, _H), lambda i: (0, i, 0)),
                  _rows(_H), _rows(1), _bcast(1, _H)],
        out_specs=[_rows(_H), _bcast(1, _H), _bcast(1, _H)],
        out_shape=[jax.ShapeDtypeStruct((_N, _H), jnp.float32), s, s],
    )(p, a, dinv, b)


def _norm(c, m, v, gm, bt, res, w, dinv):
    has_res = res is not None
    extra = [_rows(_H)] if has_res else []
    args = [c, m, v, gm, bt] + ([res] if has_res else []) + [w, dinv]
    return pl.pallas_call(
        functools.partial(_norm_body, has_res),
        grid=(_NBLK,),
        in_specs=[_rows(_H), _bcast(1, _H), _bcast(1, _H), _bcast(1, _H),
                  _bcast(1, _H)] + extra + [_bcast(_H, _H), _rows(1)],
        out_specs=[_rows(_H), _rows(_H)],
        out_shape=[jax.ShapeDtypeStruct((_N, _H), jnp.float32),
                   jax.ShapeDtypeStruct((_NP, _H), jnp.float32)],
    )(*args)


def kernel(x, edge_index, batch, W1, b1, g1, be1, W2, b2, g2, be2,
           W3, b3, g3, be3, Wfc, bfc):
    f32 = jnp.float32
    srcp = edge_index[0].reshape(_ROWS, _K)
    dstp = edge_index[1].reshape(_ROWS, _K)
    zrow_h = jnp.zeros((_RPT, _H), f32)
    zrow_d = jnp.zeros((_RPT, _DW), f32)
    ones_d = jnp.ones((_K, _DW), f32)

    degp = _deg_pass(dstp, ones_d, zrow_d)
    xw = pl.pallas_call(
        _xw_body,
        grid=(_NBLK,),
        in_specs=[_rows(_D), _bcast(_D, _H)],
        out_specs=_rows(_H),
        out_shape=jax.ShapeDtypeStruct((_NP, _H), f32),
    )(x, W1)
    dinv, a1 = pl.pallas_call(
        _scale_body,
        grid=(_NBLK,),
        in_specs=[_rows(_H),
                  pl.BlockSpec((2, _RB, _DW), lambda i: (0, i, 0))],
        out_specs=[_rows(1), _rows(_H)],
        out_shape=[jax.ShapeDtypeStruct((_N, 1), f32),
                   jax.ShapeDtypeStruct((_NP, _H), f32)],
    )(xw, degp)

    p1 = _seg_pass(a1, srcp, dstp, zrow_h)
    c1, m1, v1 = _conv(p1, a1, dinv, b1.reshape(1, _H))
    h1, a2 = _norm(c1, m1, v1, g1.reshape(1, _H), be1.reshape(1, _H),
                   None, W2, dinv)

    p2 = _seg_pass(a2, srcp, dstp, zrow_h)
    c2, m2, v2 = _conv(p2, a2, dinv, b2.reshape(1, _H))
    h2, a3 = _norm(c2, m2, v2, g2.reshape(1, _H), be2.reshape(1, _H),
                   h1, W3, dinv)

    p3 = _seg_pass(a3, srcp, dstp, zrow_h)
    c3, m3, v3 = _conv(p3, a3, dinv, b3.reshape(1, _H))
    out = pl.pallas_call(
        _pool_body,
        out_shape=jax.ShapeDtypeStruct((_G, 1), f32),
    )(c3, m3, v3, g3.reshape(1, _H), be3.reshape(1, _H),
      batch.reshape(1, _N), Wfc, bfc.reshape(1, 1))
    return out.reshape(_G)


# speedup vs baseline: 32.0121x; 1.0775x over previous
import functools

import jax
import jax.numpy as jnp
from jax import lax
from jax.experimental import pallas as pl
from jax.experimental.pallas import tpu as pltpu
from jax.experimental.pallas import tpu_sc as plsc

_N = 10000
_E = 320000
_D = 128
_H = 64
_G = 64

_NW = 32
_K = 128
_ROWS = _E // _K
_CB = _ROWS // _NW
_CX = _ROWS - _CB * _NW
_NP = 10112
_RPT = _NP // 16
_DW = 8

_mesh = plsc.VectorSubcoreMesh(core_axis_name="c", subcore_axis_name="s")


@functools.partial(
    pl.kernel,
    mesh=_mesh,
    compiler_params=pltpu.CompilerParams(use_tc_tiling_on_sc=False),
    out_type=jax.ShapeDtypeStruct((2, _NP, _DW), jnp.float32),
    scratch_types=[
        pltpu.VMEM((_CB + 1, _K), jnp.int32),
        pltpu.VMEM((_K, _DW), jnp.float32),
        pltpu.VMEM_SHARED((_NP, _DW), jnp.float32),
    ],
)
def _deg_pass(dst_hbm, ones_hbm, zrow_hbm, out_hbm, didx_v, ones_v, deg_sh):
    c = lax.axis_index("c")
    s = lax.axis_index("s")
    w = c * 16 + s
    start = w * _CB + jnp.minimum(w, _CX)
    extra = w < _CX
    pltpu.sync_copy(zrow_hbm, deg_sh.at[pl.ds(s * _RPT, _RPT)])
    pltpu.sync_copy(ones_hbm, ones_v)
    pltpu.sync_copy(dst_hbm.at[pl.ds(start, _CB)], didx_v.at[pl.ds(0, _CB)])

    @pl.when(extra)
    def _():
        pltpu.sync_copy(dst_hbm.at[pl.ds(start + _CB, 1)],
                        didx_v.at[pl.ds(_CB, 1)])

    plsc.subcore_barrier()

    def body(i, carry):
        pltpu.sync_copy(ones_v, deg_sh.at[didx_v.at[i]], add=True)
        return carry

    lax.fori_loop(0, _CB + extra.astype(jnp.int32), body, 0)
    plsc.subcore_barrier()
    pltpu.sync_copy(deg_sh.at[pl.ds(s * _RPT, _RPT)],
                    out_hbm.at[c, pl.ds(s * _RPT, _RPT)])


@functools.partial(
    pl.kernel,
    mesh=_mesh,
    compiler_params=pltpu.CompilerParams(use_tc_tiling_on_sc=False),
    out_type=jax.ShapeDtypeStruct((_NP, 2 * _H), jnp.float32),
    scratch_types=[
        pltpu.VMEM((_CB + 1, _K), jnp.int32),
        pltpu.VMEM((_CB + 1, _K), jnp.int32),
        pltpu.VMEM((_K, _H), jnp.float32),
        pltpu.VMEM((_K, _H), jnp.float32),
        pltpu.VMEM_SHARED((_NP, _H), jnp.float32),
        pltpu.VMEM_SHARED((_NP, _H), jnp.float32),
        pltpu.SemaphoreType.DMA,
        pltpu.SemaphoreType.DMA,
    ],
)
def _seg_pass(a_hbm, src_hbm, dst_hbm, zrow_hbm, out_hbm,
              sidx_v, didx_v, buf0, buf1, agg_sh, feat_sh, sem0, sem1):
    c = lax.axis_index("c")
    s = lax.axis_index("s")
    w = c * 16 + s
    start = w * _CB + jnp.minimum(w, _CX)
    extra = w < _CX
    pltpu.sync_copy(zrow_hbm, agg_sh.at[pl.ds(s * _RPT, _RPT)])
    pltpu.sync_copy(a_hbm.at[pl.ds(s * _RPT, _RPT)],
                    feat_sh.at[pl.ds(s * _RPT, _RPT)])
    pltpu.sync_copy(src_hbm.at[pl.ds(start, _CB)], sidx_v.at[pl.ds(0, _CB)])
    pltpu.sync_copy(dst_hbm.at[pl.ds(start, _CB)], didx_v.at[pl.ds(0, _CB)])

    @pl.when(extra)
    def _():
        pltpu.sync_copy(src_hbm.at[pl.ds(start + _CB, 1)],
                        sidx_v.at[pl.ds(_CB, 1)])
        pltpu.sync_copy(dst_hbm.at[pl.ds(start + _CB, 1)],
                        didx_v.at[pl.ds(_CB, 1)])

    plsc.subcore_barrier()
    g0 = pltpu.async_copy(feat_sh.at[sidx_v.at[0]], buf0, sem0)

    def body(j, carry):
        i0 = 2 * j
        g1 = pltpu.async_copy(feat_sh.at[sidx_v.at[i0 + 1]], buf1, sem1)
        pltpu.make_async_copy(feat_sh.at[pl.ds(0, _K)], buf0, sem0).wait()
        pltpu.sync_copy(buf0, agg_sh.at[didx_v.at[i0]], add=True)

        @pl.when(j < _CB // 2 - 1)
        def _():
            pltpu.async_copy(feat_sh.at[sidx_v.at[i0 + 2]], buf0, sem0)

        g1.wait()
        pltpu.sync_copy(buf1, agg_sh.at[didx_v.at[i0 + 1]], add=True)
        return carry

    lax.fori_loop(0, _CB // 2, body, 0)

    @pl.when(extra)
    def _():
        pltpu.sync_copy(feat_sh.at[sidx_v.at[_CB]], buf0)
        pltpu.sync_copy(buf0, agg_sh.at[didx_v.at[_CB]], add=True)

    plsc.subcore_barrier()
    pltpu.sync_copy(agg_sh.at[pl.ds(s * _RPT, _RPT)],
                    out_hbm.at[pl.ds(s * _RPT, _RPT), pl.ds(c * _H, _H)])


_RB = 2528
_NBLK = _NP // _RB


def _xw_body(x_ref, w_ref, xw_ref):
    xw_ref[...] = jnp.dot(x_ref[...], w_ref[...],
                          preferred_element_type=jnp.float32,
                          precision=lax.Precision.HIGHEST)


def _scale_body(xw_ref, dp_ref, dinv_ref, a_ref):
    deg = dp_ref[0, :, 0:1] + dp_ref[1, :, 0:1] + 1.0
    dinv = lax.rsqrt(deg)
    dinv_ref[...] = dinv
    a_ref[...] = dinv * xw_ref[...]


def _conv_body(p_ref, a_ref, dinv_ref, b_ref, conv_ref, m_ref, v_ref):
    i = pl.program_id(0)
    c = (dinv_ref[...] * (p_ref[:, :_H] + p_ref[:, _H:] + a_ref[...])
         + b_ref[...])
    conv_ref[...] = c
    rows = lax.broadcasted_iota(jnp.int32, (_RB, 1), 0) + i * _RB
    cm = jnp.where(rows < _N, c, 0.0)

    @pl.when(i == 0)
    def _():
        m_ref[...] = jnp.zeros_like(m_ref)
        v_ref[...] = jnp.zeros_like(v_ref)

    m_ref[...] += jnp.sum(cm, axis=0, keepdims=True)
    v_ref[...] += jnp.sum(cm * cm, axis=0, keepdims=True)

    @pl.when(i == _NBLK - 1)
    def _():
        m = m_ref[...] / _N
        m_ref[...] = m
        v_ref[...] = v_ref[...] / _N - m * m


def _norm_body(has_res, c_ref, m_ref, v_ref, gm_ref, bt_ref, *rest):
    if has_res:
        res_ref, w_ref, dinv_ref, h_ref, an_ref = rest
    else:
        w_ref, dinv_ref, h_ref, an_ref = rest
    h = ((c_ref[...] - m_ref[...]) / jnp.sqrt(v_ref[...] + 1e-5)
         * gm_ref[...] + bt_ref[...])
    if has_res:
        h = h + res_ref[...]
    h = jnp.maximum(h, 0.0)
    h_ref[...] = h
    hw = jnp.dot(h, w_ref[...], preferred_element_type=jnp.float32,
                 precision=lax.Precision.HIGHEST)
    an_ref[...] = dinv_ref[...] * hw


def _pool_body(c_ref, m_ref, v_ref, gm_ref, bt_ref, batch_ref,
               wfc_ref, bfc_ref, out_ref):
    h = ((c_ref[...] - m_ref[...]) / jnp.sqrt(v_ref[...] + 1e-5)
         * gm_ref[...] + bt_ref[...])
    h = jnp.maximum(h, 0.0)
    gids = lax.broadcasted_iota(jnp.int32, (_G, _N), 0)
    onehot = (gids == batch_ref[...]).astype(jnp.float32)
    counts = jnp.sum(onehot, axis=1, keepdims=True)
    pooled = jnp.dot(onehot, h, preferred_element_type=jnp.float32,
                     precision=lax.Precision.HIGHEST)
    pooled = pooled / jnp.maximum(counts, 1.0)
    out_ref[...] = jnp.dot(pooled, wfc_ref[...],
                           preferred_element_type=jnp.float32,
                           precision=lax.Precision.HIGHEST) + bfc_ref[...]


def _rows(bs):
    return pl.BlockSpec((_RB, bs), lambda i: (i, 0))


def _bcast(r, c):
    return pl.BlockSpec((r, c), lambda i: (0, 0))


def _conv(p, a, dinv, b):
    s = jax.ShapeDtypeStruct((1, _H), jnp.float32)
    return pl.pallas_call(
        _conv_body,
        grid=(_NBLK,),
        in_specs=[pl.BlockSpec((_RB, 2 * _H), lambda i: (i, 0)),
                  _rows(_H), _rows(1), _bcast(1, _H)],
        out_specs=[_rows(_H), _bcast(1, _H), _bcast(1, _H)],
        out_shape=[jax.ShapeDtypeStruct((_N, _H), jnp.float32), s, s],
    )(p, a, dinv, b)


def _norm(c, m, v, gm, bt, res, w, dinv):
    has_res = res is not None
    extra = [_rows(_H)] if has_res else []
    args = [c, m, v, gm, bt] + ([res] if has_res else []) + [w, dinv]
    return pl.pallas_call(
        functools.partial(_norm_body, has_res),
        grid=(_NBLK,),
        in_specs=[_rows(_H), _bcast(1, _H), _bcast(1, _H), _bcast(1, _H),
                  _bcast(1, _H)] + extra + [_bcast(_H, _H), _rows(1)],
        out_specs=[_rows(_H), _rows(_H)],
        out_shape=[jax.ShapeDtypeStruct((_N, _H), jnp.float32),
                   jax.ShapeDtypeStruct((_NP, _H), jnp.float32)],
    )(*args)


def kernel(x, edge_index, batch, W1, b1, g1, be1, W2, b2, g2, be2,
           W3, b3, g3, be3, Wfc, bfc):
    f32 = jnp.float32
    srcp = edge_index[0].reshape(_ROWS, _K)
    dstp = edge_index[1].reshape(_ROWS, _K)
    zrow_h = jnp.zeros((_RPT, _H), f32)
    zrow_d = jnp.zeros((_RPT, _DW), f32)
    ones_d = jnp.ones((_K, _DW), f32)

    degp = _deg_pass(dstp, ones_d, zrow_d)
    xw = pl.pallas_call(
        _xw_body,
        grid=(_NBLK,),
        in_specs=[_rows(_D), _bcast(_D, _H)],
        out_specs=_rows(_H),
        out_shape=jax.ShapeDtypeStruct((_NP, _H), f32),
    )(x, W1)
    dinv, a1 = pl.pallas_call(
        _scale_body,
        grid=(_NBLK,),
        in_specs=[_rows(_H),
                  pl.BlockSpec((2, _RB, _DW), lambda i: (0, i, 0))],
        out_specs=[_rows(1), _rows(_H)],
        out_shape=[jax.ShapeDtypeStruct((_N, 1), f32),
                   jax.ShapeDtypeStruct((_NP, _H), f32)],
    )(xw, degp)

    p1 = _seg_pass(a1, srcp, dstp, zrow_h)
    c1, m1, v1 = _conv(p1, a1, dinv, b1.reshape(1, _H))
    h1, a2 = _norm(c1, m1, v1, g1.reshape(1, _H), be1.reshape(1, _H),
                   None, W2, dinv)

    p2 = _seg_pass(a2, srcp, dstp, zrow_h)
    c2, m2, v2 = _conv(p2, a2, dinv, b2.reshape(1, _H))
    h2, a3 = _norm(c2, m2, v2, g2.reshape(1, _H), be2.reshape(1, _H),
                   h1, W3, dinv)

    p3 = _seg_pass(a3, srcp, dstp, zrow_h)
    c3, m3, v3 = _conv(p3, a3, dinv, b3.reshape(1, _H))
    out = pl.pallas_call(
        _pool_body,
        out_shape=jax.ShapeDtypeStruct((_G, 1), f32),
    )(c3, m3, v3, g3.reshape(1, _H), be3.reshape(1, _H),
      batch.reshape(1, _N), Wfc, bfc.reshape(1, 1))
    return out.reshape(_G)


# speedup vs baseline: 33.3541x; 1.0419x over previous
import functools

import jax
import jax.numpy as jnp
from jax import lax
from jax.experimental import pallas as pl
from jax.experimental.pallas import tpu as pltpu
from jax.experimental.pallas import tpu_sc as plsc

_N = 10000
_E = 320000
_D = 128
_H = 64
_G = 64

_NW = 32
_K = 128
_ROWS = _E // _K
_CB = _ROWS // _NW
_CX = _ROWS - _CB * _NW
_NP = 10112
_RPT = _NP // 16
_DW = 8

_mesh = plsc.VectorSubcoreMesh(core_axis_name="c", subcore_axis_name="s")


@functools.partial(
    pl.kernel,
    mesh=_mesh,
    compiler_params=pltpu.CompilerParams(use_tc_tiling_on_sc=False),
    out_type=jax.ShapeDtypeStruct((2, _NP, _DW), jnp.float32),
    scratch_types=[
        pltpu.VMEM((_CB + 1, _K), jnp.int32),
        pltpu.VMEM((_K, _DW), jnp.float32),
        pltpu.VMEM_SHARED((_NP, _DW), jnp.float32),
    ],
)
def _deg_pass(dst_hbm, ones_hbm, zrow_hbm, out_hbm, didx_v, ones_v, deg_sh):
    c = lax.axis_index("c")
    s = lax.axis_index("s")
    w = c * 16 + s
    start = w * _CB + jnp.minimum(w, _CX)
    extra = w < _CX
    pltpu.sync_copy(zrow_hbm, deg_sh.at[pl.ds(s * _RPT, _RPT)])
    pltpu.sync_copy(ones_hbm, ones_v)
    pltpu.sync_copy(dst_hbm.at[pl.ds(start, _CB)], didx_v.at[pl.ds(0, _CB)])

    @pl.when(extra)
    def _():
        pltpu.sync_copy(dst_hbm.at[pl.ds(start + _CB, 1)],
                        didx_v.at[pl.ds(_CB, 1)])

    plsc.subcore_barrier()

    def body(i, carry):
        pltpu.sync_copy(ones_v, deg_sh.at[didx_v.at[i]], add=True)
        return carry

    lax.fori_loop(0, _CB + extra.astype(jnp.int32), body, 0)
    plsc.subcore_barrier()
    pltpu.sync_copy(deg_sh.at[pl.ds(s * _RPT, _RPT)],
                    out_hbm.at[c, pl.ds(s * _RPT, _RPT)])


@functools.partial(
    pl.kernel,
    mesh=_mesh,
    compiler_params=pltpu.CompilerParams(use_tc_tiling_on_sc=False),
    out_type=jax.ShapeDtypeStruct((_NP, 2 * _H), jnp.float32),
    scratch_types=[
        pltpu.VMEM((_CB + 1, _K), jnp.int32),
        pltpu.VMEM((_CB + 1, _K), jnp.int32),
        pltpu.VMEM((_K, _H), jnp.float32),
        pltpu.VMEM((_K, _H), jnp.float32),
        pltpu.VMEM_SHARED((_NP, _H), jnp.float32),
        pltpu.VMEM_SHARED((_NP, _H), jnp.float32),
        pltpu.SemaphoreType.DMA,
        pltpu.SemaphoreType.DMA,
    ],
)
def _seg_pass(a_hbm, src_hbm, dst_hbm, zrow_hbm, out_hbm,
              sidx_v, didx_v, buf0, buf1, agg_sh, feat_sh, sem0, sem1):
    c = lax.axis_index("c")
    s = lax.axis_index("s")
    w = c * 16 + s
    start = w * _CB + jnp.minimum(w, _CX)
    extra = w < _CX
    pltpu.sync_copy(zrow_hbm, agg_sh.at[pl.ds(s * _RPT, _RPT)])
    pltpu.sync_copy(a_hbm.at[pl.ds(s * _RPT, _RPT), pl.ds(0, _H)],
                    feat_sh.at[pl.ds(s * _RPT, _RPT)])
    pltpu.sync_copy(src_hbm.at[pl.ds(start, _CB)], sidx_v.at[pl.ds(0, _CB)])
    pltpu.sync_copy(dst_hbm.at[pl.ds(start, _CB)], didx_v.at[pl.ds(0, _CB)])

    @pl.when(extra)
    def _():
        pltpu.sync_copy(src_hbm.at[pl.ds(start + _CB, 1)],
                        sidx_v.at[pl.ds(_CB, 1)])
        pltpu.sync_copy(dst_hbm.at[pl.ds(start + _CB, 1)],
                        didx_v.at[pl.ds(_CB, 1)])

    plsc.subcore_barrier()
    g0 = pltpu.async_copy(feat_sh.at[sidx_v.at[0]], buf0, sem0)

    def body(j, carry):
        i0 = 2 * j
        g1 = pltpu.async_copy(feat_sh.at[sidx_v.at[i0 + 1]], buf1, sem1)
        pltpu.make_async_copy(feat_sh.at[pl.ds(0, _K)], buf0, sem0).wait()
        pltpu.sync_copy(buf0, agg_sh.at[didx_v.at[i0]], add=True)

        @pl.when(j < _CB // 2 - 1)
        def _():
            pltpu.async_copy(feat_sh.at[sidx_v.at[i0 + 2]], buf0, sem0)

        g1.wait()
        pltpu.sync_copy(buf1, agg_sh.at[didx_v.at[i0 + 1]], add=True)
        return carry

    lax.fori_loop(0, _CB // 2, body, 0)

    @pl.when(extra)
    def _():
        pltpu.sync_copy(feat_sh.at[sidx_v.at[_CB]], buf0)
        pltpu.sync_copy(buf0, agg_sh.at[didx_v.at[_CB]], add=True)

    plsc.subcore_barrier()
    pltpu.sync_copy(agg_sh.at[pl.ds(s * _RPT, _RPT)],
                    out_hbm.at[pl.ds(s * _RPT, _RPT), pl.ds(c * _H, _H)])


_RB = 2528
_NBLK = _NP // _RB


def _xw_body(x_ref, w_ref, xw_ref):
    xw_ref[...] = jnp.dot(x_ref[...], w_ref[...],
                          preferred_element_type=jnp.float32,
                          precision=lax.Precision.HIGHEST)


def _scale_body(xw_ref, dp_ref, dinv_ref, a_ref):
    deg = dp_ref[0, :, 0:1] + dp_ref[1, :, 0:1] + 1.0
    dinv = lax.rsqrt(deg)
    dinv_ref[...] = dinv
    a = dinv * xw_ref[...]
    a_ref[...] = jnp.concatenate([a, jnp.zeros_like(a)], axis=1)


def _conv_body(p_ref, a_ref, dinv_ref, b_ref, conv_ref, m_ref, v_ref):
    i = pl.program_id(0)
    c = (dinv_ref[...] * (p_ref[:, :_H] + p_ref[:, _H:] + a_ref[:, :_H])
         + b_ref[...])
    conv_ref[...] = c
    rows = lax.broadcasted_iota(jnp.int32, (_RB, 1), 0) + i * _RB
    cm = jnp.where(rows < _N, c, 0.0)

    @pl.when(i == 0)
    def _():
        m_ref[...] = jnp.zeros_like(m_ref)
        v_ref[...] = jnp.zeros_like(v_ref)

    m_ref[...] += jnp.sum(cm, axis=0, keepdims=True)
    v_ref[...] += jnp.sum(cm * cm, axis=0, keepdims=True)

    @pl.when(i == _NBLK - 1)
    def _():
        m = m_ref[...] / _N
        m_ref[...] = m
        v_ref[...] = v_ref[...] / _N - m * m


def _norm_body(has_res, c_ref, m_ref, v_ref, gm_ref, bt_ref, *rest):
    if has_res:
        res_ref, w_ref, dinv_ref, out_ref = rest
    else:
        w_ref, dinv_ref, out_ref = rest
    h = ((c_ref[...] - m_ref[...]) / jnp.sqrt(v_ref[...] + 1e-5)
         * gm_ref[...] + bt_ref[...])
    if has_res:
        h = h + res_ref[:, _H:]
    h = jnp.maximum(h, 0.0)
    hw = jnp.dot(h, w_ref[...], preferred_element_type=jnp.float32,
                 precision=lax.Precision.HIGHEST)
    out_ref[...] = jnp.concatenate([dinv_ref[...] * hw, h], axis=1)


def _pool_body(c_ref, m_ref, v_ref, gm_ref, bt_ref, batch_ref,
               wfc_ref, bfc_ref, out_ref):
    h = ((c_ref[...] - m_ref[...]) / jnp.sqrt(v_ref[...] + 1e-5)
         * gm_ref[...] + bt_ref[...])
    h = jnp.maximum(h, 0.0)
    gids = lax.broadcasted_iota(jnp.int32, (_G, _N), 0)
    onehot = (gids == batch_ref[...]).astype(jnp.float32)
    counts = jnp.sum(onehot, axis=1, keepdims=True)
    pooled = jnp.dot(onehot, h, preferred_element_type=jnp.float32,
                     precision=lax.Precision.HIGHEST)
    pooled = pooled / jnp.maximum(counts, 1.0)
    out_ref[...] = jnp.dot(pooled, wfc_ref[...],
                           preferred_element_type=jnp.float32,
                           precision=lax.Precision.HIGHEST) + bfc_ref[...]


def _rows(bs):
    return pl.BlockSpec((_RB, bs), lambda i: (i, 0))


def _bcast(r, c):
    return pl.BlockSpec((r, c), lambda i: (0, 0))


def _conv(p, a, dinv, b):
    s = jax.ShapeDtypeStruct((1, _H), jnp.float32)
    return pl.pallas_call(
        _conv_body,
        grid=(_NBLK,),
        in_specs=[pl.BlockSpec((_RB, 2 * _H), lambda i: (i, 0)),
                  _rows(2 * _H), _rows(1), _bcast(1, _H)],
        out_specs=[_rows(_H), _bcast(1, _H), _bcast(1, _H)],
        out_shape=[jax.ShapeDtypeStruct((_N, _H), jnp.float32), s, s],
    )(p, a, dinv, b)


def _norm(c, m, v, gm, bt, res, w, dinv):
    has_res = res is not None
    extra = [_rows(2 * _H)] if has_res else []
    args = [c, m, v, gm, bt] + ([res] if has_res else []) + [w, dinv]
    return pl.pallas_call(
        functools.partial(_norm_body, has_res),
        grid=(_NBLK,),
        in_specs=[_rows(_H), _bcast(1, _H), _bcast(1, _H), _bcast(1, _H),
                  _bcast(1, _H)] + extra + [_bcast(_H, _H), _rows(1)],
        out_specs=_rows(2 * _H),
        out_shape=jax.ShapeDtypeStruct((_NP, 2 * _H), jnp.float32),
    )(*args)


def kernel(x, edge_index, batch, W1, b1, g1, be1, W2, b2, g2, be2,
           W3, b3, g3, be3, Wfc, bfc):
    f32 = jnp.float32
    srcp = edge_index[0].reshape(_ROWS, _K)
    dstp = edge_index[1].reshape(_ROWS, _K)
    zrow_h = jnp.zeros((_RPT, _H), f32)
    zrow_d = jnp.zeros((_RPT, _DW), f32)
    ones_d = jnp.ones((_K, _DW), f32)

    degp = _deg_pass(dstp, ones_d, zrow_d)
    xw = pl.pallas_call(
        _xw_body,
        grid=(_NBLK,),
        in_specs=[_rows(_D), _bcast(_D, _H)],
        out_specs=_rows(_H),
        out_shape=jax.ShapeDtypeStruct((_NP, _H), f32),
    )(x, W1)
    dinv, a1 = pl.pallas_call(
        _scale_body,
        grid=(_NBLK,),
        in_specs=[_rows(_H),
                  pl.BlockSpec((2, _RB, _DW), lambda i: (0, i, 0))],
        out_specs=[_rows(1), _rows(2 * _H)],
        out_shape=[jax.ShapeDtypeStruct((_N, 1), f32),
                   jax.ShapeDtypeStruct((_NP, 2 * _H), f32)],
    )(xw, degp)

    p1 = _seg_pass(a1, srcp, dstp, zrow_h)
    c1, m1, v1 = _conv(p1, a1, dinv, b1.reshape(1, _H))
    a2 = _norm(c1, m1, v1, g1.reshape(1, _H), be1.reshape(1, _H),
               None, W2, dinv)

    p2 = _seg_pass(a2, srcp, dstp, zrow_h)
    c2, m2, v2 = _conv(p2, a2, dinv, b2.reshape(1, _H))
    a3 = _norm(c2, m2, v2, g2.reshape(1, _H), be2.reshape(1, _H),
               a2, W3, dinv)

    p3 = _seg_pass(a3, srcp, dstp, zrow_h)
    c3, m3, v3 = _conv(p3, a3, dinv, b3.reshape(1, _H))
    out = pl.pallas_call(
        _pool_body,
        out_shape=jax.ShapeDtypeStruct((_G, 1), f32),
    )(c3, m3, v3, g3.reshape(1, _H), be3.reshape(1, _H),
      batch.reshape(1, _N), Wfc, bfc.reshape(1, 1))
    return out.reshape(_G)
